# Initial kernel scaffold; baseline (speedup 1.0000x reference)
import functools
import math

import jax
import jax.numpy as jnp
from jax import lax
from jax.experimental import pallas as pl
from jax.experimental.pallas import tpu as pltpu
from jax.experimental.pallas import tpu_sc as plsc

N = 10000
B = 100
L = 100
D = 128
E = 320000
R = 4
NSTEP = 2
NR = N * R

NC = 2
NS = 16
NW = NC * NS
EPW = E // NW
CH = 80
NCH = EPW // CH
RPT = N // NS
CRP = NR // NS


def _enc_body(x_ref, w_ref, b_ref, o_ref):
    y = jnp.dot(x_ref[...], w_ref[...], preferred_element_type=jnp.float32)
    o_ref[...] = jnp.maximum(y + b_ref[...], 0.0)


def _enc(x, w_t, b):
    blk = 2000
    return pl.pallas_call(
        _enc_body,
        grid=(N // blk,),
        in_specs=[
            pl.BlockSpec((blk, D), lambda i: (i, 0)),
            pl.BlockSpec((D, D), lambda i: (0, 0)),
            pl.BlockSpec((1, D), lambda i: (0, 0)),
        ],
        out_specs=pl.BlockSpec((blk, D), lambda i: (i, 0)),
        out_shape=jax.ShapeDtypeStruct((N, D), jnp.float32),
    )(x, w_t, b.reshape(1, D))


def _inv_body(c0_ref, c1_ref, o_ref):
    s = c0_ref[:, 0:1] + c1_ref[:, 0:1]
    inv = 1.0 / jnp.maximum(s, 1.0)
    o_ref[...] = jnp.broadcast_to(inv, o_ref.shape)


def _prep_inv(cnt0, cnt1):
    blk = 4000
    return pl.pallas_call(
        _inv_body,
        grid=(NR // blk,),
        in_specs=[
            pl.BlockSpec((blk, 16), lambda i: (i, 0)),
            pl.BlockSpec((blk, 16), lambda i: (i, 0)),
        ],
        out_specs=pl.BlockSpec((blk, 2 * D), lambda i: (i, 0)),
        out_shape=jax.ShapeDtypeStruct((NR, 2 * D), jnp.float32),
    )(cnt0, cnt1)


def _pre_body(h_ref, w_ref, bias_ref, inv_ref, skip_ref, xl_ref, gb_ref):
    y = jnp.dot(h_ref[...], w_ref[...], preferred_element_type=jnp.float32)
    skip_ref[...] = jnp.maximum(y[:, 256:384] * y[:, 0:128] + y[:, 128:256], 0.0)
    xl_ref[...] = y[:, 384:896]
    gb_ref[...] = (y[:, 896:1920] + bias_ref[...]) * inv_ref[...]


def _pre(h, w_cat, bias_cat, inv_n):
    blk = 1000
    return pl.pallas_call(
        _pre_body,
        grid=(N // blk,),
        in_specs=[
            pl.BlockSpec((blk, D), lambda i: (i, 0)),
            pl.BlockSpec((D, 15 * D), lambda i: (0, 0)),
            pl.BlockSpec((1, 8 * D), lambda i: (0, 0)),
            pl.BlockSpec((blk, 8 * D), lambda i: (i, 0)),
        ],
        out_specs=[
            pl.BlockSpec((blk, D), lambda i: (i, 0)),
            pl.BlockSpec((blk, 4 * D), lambda i: (i, 0)),
            pl.BlockSpec((blk, 8 * D), lambda i: (i, 0)),
        ],
        out_shape=[
            jax.ShapeDtypeStruct((N, D), jnp.float32),
            jax.ShapeDtypeStruct((N, 4 * D), jnp.float32),
            jax.ShapeDtypeStruct((N, 8 * D), jnp.float32),
        ],
    )(h, w_cat, bias_cat, inv_n)


def _attn_body(h_ref, m_ref, q_ref):
    hm = h_ref[0] * m_ref[0]
    logits = lax.dot_general(
        hm, hm, (((1,), (1,)), ((), ())), preferred_element_type=jnp.float32
    ) * (1.0 / math.sqrt(D))
    mx = jnp.max(logits, axis=-1, keepdims=True)
    e = jnp.exp(logits - mx)
    aw = e / jnp.sum(e, axis=-1, keepdims=True)
    a = jnp.dot(aw, hm, preferred_element_type=jnp.float32)
    q_ref[0, 0] = jnp.mean(a, axis=0)


def _attn(h3, mask):
    out = pl.pallas_call(
        _attn_body,
        grid=(B,),
        in_specs=[
            pl.BlockSpec((1, L, D), lambda i: (i, 0, 0)),
            pl.BlockSpec((1, L, 1), lambda i: (i, 0, 0)),
        ],
        out_specs=pl.BlockSpec((1, 1, D), lambda i: (i, 0, 0)),
        out_shape=jax.ShapeDtypeStruct((B, 1, D), jnp.float32),
    )(h3, mask)
    return out.reshape(B, D)


def _gru_body(h_ref, skip_ref, p0_ref, p1_ref, sn_ref, wa_ref, wx_ref,
              ws_ref, wh1_ref, bias_ref, o_ref):
    h = h_ref[...]
    a = skip_ref[...] + p0_ref[...] + p1_ref[...]
    sn = sn_ref[...]
    ya = jnp.dot(a, wa_ref[...], preferred_element_type=jnp.float32)
    yx = jnp.dot(h, wx_ref[...], preferred_element_type=jnp.float32)
    ys = jnp.dot(sn, ws_ref[...], preferred_element_type=jnp.float32)
    bias = bias_ref[...]
    z = jax.nn.sigmoid(ya[:, 0:D] + yx[:, 0:D] + ys[:, 0:D] + bias[:, 0:D])
    r = jax.nn.sigmoid(
        ya[:, D:2 * D] + yx[:, D:2 * D] + ys[:, D:2 * D] + bias[:, D:2 * D])
    hr = jnp.dot(h * r, wh1_ref[...], preferred_element_type=jnp.float32)
    hh = jnp.maximum(
        ya[:, 2 * D:3 * D] + hr + ys[:, 2 * D:3 * D] + bias[:, 2 * D:3 * D], 0.0)
    o_ref[...] = hh * z + h * (1.0 - z)


def _gru(h, skip, p0, p1, sn, wa, wx, ws, wh1, bias):
    blk = 2000
    return pl.pallas_call(
        _gru_body,
        grid=(N // blk,),
        in_specs=[
            pl.BlockSpec((blk, D), lambda i: (i, 0)),
            pl.BlockSpec((blk, D), lambda i: (i, 0)),
            pl.BlockSpec((blk, D), lambda i: (i, 0)),
            pl.BlockSpec((blk, D), lambda i: (i, 0)),
            pl.BlockSpec((blk, D), lambda i: (i, 0)),
            pl.BlockSpec((D, 3 * D), lambda i: (0, 0)),
            pl.BlockSpec((D, 2 * D), lambda i: (0, 0)),
            pl.BlockSpec((D, 3 * D), lambda i: (0, 0)),
            pl.BlockSpec((D, D), lambda i: (0, 0)),
            pl.BlockSpec((1, 3 * D), lambda i: (0, 0)),
        ],
        out_specs=pl.BlockSpec((blk, D), lambda i: (i, 0)),
        out_shape=jax.ShapeDtypeStruct((N, D), jnp.float32),
    )(h, skip, p0, p1, sn, wa, wx, ws, wh1, bias)


def _count_sc(dstr):
    mesh = plsc.VectorSubcoreMesh(core_axis_name="c", subcore_axis_name="s")

    @functools.partial(
        pl.kernel,
        out_type=jax.ShapeDtypeStruct((NC, NR, 16), jnp.float32),
        mesh=mesh,
        scratch_types=[
            pltpu.VMEM((CH,), jnp.int32),
            pltpu.VMEM((CH, 16), jnp.float32),
            pltpu.VMEM((100, 16), jnp.float32),
            pltpu.VMEM_SHARED((NR, 16), jnp.float32),
        ],
    )
    def k(dstr_hbm, out_hbm, ibuf, ones_v, zbuf, cnt_sh):
        c = lax.axis_index("c")
        s = lax.axis_index("s")
        wid = s * NC + c
        lanes = lax.iota(jnp.int32, 16)
        unit = jnp.where(lanes == 0, 1.0, 0.0).astype(jnp.float32)
        zero = jnp.zeros((16,), jnp.float32)

        def fill(i, _):
            ones_v[i, :] = unit
            return 0

        lax.fori_loop(0, CH, fill, 0)

        def fillz(i, _):
            zbuf[i, :] = zero
            return 0

        lax.fori_loop(0, 100, fillz, 0)

        def zcp(i, _):
            pltpu.sync_copy(zbuf, cnt_sh.at[pl.ds(s * CRP + i * 100, 100)])
            return 0

        lax.fori_loop(0, CRP // 100, zcp, 0)
        plsc.subcore_barrier()

        def chunk(i, _):
            base = wid * EPW + i * CH
            pltpu.sync_copy(dstr_hbm.at[pl.ds(base, CH)], ibuf)
            pltpu.sync_copy(ones_v, cnt_sh.at[ibuf], add=True)
            return 0

        lax.fori_loop(0, NCH, chunk, 0)
        plsc.subcore_barrier()

        def dump(i, _):
            sl = pl.ds(s * CRP + i * 100, 100)
            pltpu.sync_copy(cnt_sh.at[sl], out_hbm.at[c, sl])
            return 0

        lax.fori_loop(0, CRP // 100, dump, 0)

    return k(dstr)


def _edges_sc(xl_tab, gb_tab, srcr, dstr):
    mesh = plsc.VectorSubcoreMesh(core_axis_name="c", subcore_axis_name="s")

    @functools.partial(
        pl.kernel,
        out_type=jax.ShapeDtypeStruct((NC, N, D), jnp.float32),
        mesh=mesh,
        scratch_types=[
            pltpu.VMEM((CH,), jnp.int32),
            pltpu.VMEM((CH,), jnp.int32),
            pltpu.VMEM((CH,), jnp.int32),
            pltpu.VMEM((CH, D), jnp.float32),
            pltpu.VMEM((CH, 2 * D), jnp.float32),
            pltpu.VMEM((CH, D), jnp.float32),
            pltpu.VMEM_SHARED((N, D), jnp.float32),
            pltpu.SemaphoreType.DMA,
            pltpu.SemaphoreType.DMA,
        ],
    )
    def k(xl_hbm, gb_hbm, srcr_hbm, dstr_hbm, out_hbm,
          isrc, idst, dstb, xbuf, gbbuf, msgbuf, accum, sem1, sem2):
        c = lax.axis_index("c")
        s = lax.axis_index("s")
        wid = s * NC + c
        ebase = wid * EPW
        zero = jnp.zeros((16,), jnp.float32)

        def zb(i, _):
            for j in range(D // 16):
                msgbuf[i, pl.ds(j * 16, 16)] = zero
            return 0

        lax.fori_loop(0, CH, zb, 0)
        rbase = s * RPT
        for i in range(RPT // CH):
            pltpu.sync_copy(msgbuf, accum.at[pl.ds(rbase + i * CH, CH)])
        rem = RPT - (RPT // CH) * CH
        pltpu.sync_copy(msgbuf.at[pl.ds(0, rem)],
                        accum.at[pl.ds(rbase + RPT - rem, rem)])
        plsc.subcore_barrier()

        def chunk(ci, _):
            base = ebase + ci * CH
            pltpu.sync_copy(srcr_hbm.at[pl.ds(base, CH)], isrc)
            pltpu.sync_copy(dstr_hbm.at[pl.ds(base, CH)], idst)
            g1 = pltpu.async_copy(xl_hbm.at[isrc], xbuf, sem1)
            g2 = pltpu.async_copy(gb_hbm.at[idst], gbbuf, sem2)
            for j in range(CH // 16):
                dstb[pl.ds(j * 16, 16)] = lax.shift_right_logical(
                    idst[pl.ds(j * 16, 16)], 2)
            g1.wait()
            g2.wait()

            def edge(e, _):
                for j in range(D // 16):
                    g = gbbuf[e, pl.ds(D + j * 16, 16)]
                    bt = gbbuf[e, pl.ds(j * 16, 16)]
                    xv = xbuf[e, pl.ds(j * 16, 16)]
                    msgbuf[e, pl.ds(j * 16, 16)] = jnp.maximum(g * xv + bt, 0.0)
                return 0

            lax.fori_loop(0, CH, edge, 0)
            pltpu.sync_copy(msgbuf, accum.at[dstb], add=True)
            return 0

        lax.fori_loop(0, NCH, chunk, 0)
        plsc.subcore_barrier()
        pltpu.sync_copy(accum.at[pl.ds(rbase, RPT)],
                        out_hbm.at[c, pl.ds(rbase, RPT)])

    return k(xl_tab, gb_tab, srcr, dstr)


def kernel(x, edge_index, edge_type, mask, params):
    src = edge_index[0].astype(jnp.int32)
    dst = edge_index[1].astype(jnp.int32)
    rt = edge_type.astype(jnp.int32)
    srcr = src * R + rt
    dstr = dst * R + rt

    cnt = _count_sc(dstr)
    inv_exp = _prep_inv(cnt[0], cnt[1])
    inv_n = inv_exp.reshape(N, 8 * D)

    p = params
    h = _enc(x, p["enc"]["W"].T, p["enc"]["b"])

    gp = p["gru_s"]
    wa = jnp.concatenate(
        [gp["z0"]["W"].T, gp["r0"]["W"].T, gp["h0"]["W"].T], axis=1)
    wx = jnp.concatenate([gp["z1"]["W"].T, gp["r1"]["W"].T], axis=1)
    ws = jnp.concatenate(
        [gp["z2"]["W"].T, gp["r2"]["W"].T, gp["h2"]["W"].T], axis=1)
    wh1 = gp["h1"]["W"].T
    gbias = jnp.concatenate([
        gp["z0"]["b"] + gp["z1"]["b"] + gp["z2"]["b"],
        gp["r0"]["b"] + gp["r1"]["b"] + gp["r2"]["b"],
        gp["h0"]["b"] + gp["h1"]["b"] + gp["h2"]["b"],
    ]).reshape(1, 3 * D)

    for step in range(NSTEP):
        cp = p["conv"][step]
        w_cat = jnp.concatenate(
            [cp["lin_skip"]["W"].T, cp["film_skip"]["W"].T]
            + [cp["lins"][r]["W"].T for r in range(R)]
            + [cp["films"][r]["W"].T for r in range(R)],
            axis=1)
        bias_cat = jnp.concatenate(
            [cp["films"][r]["b"] for r in range(R)]).reshape(1, 8 * D)

        skip, xl, gb = _pre(h, w_cat, bias_cat, inv_n)
        xl_tab = xl.reshape(NR, D)
        gb_tab = gb.reshape(NR, 2 * D)

        q = _attn(h.reshape(B, L, D), mask)
        sn = jnp.repeat(q, L, axis=0)

        parts = _edges_sc(xl_tab, gb_tab, srcr, dstr)
        h = _gru(h, skip, parts[0], parts[1], sn, wa, wx, ws, wh1, gbias)

    return h.reshape(B, L, D)


# speedup vs baseline: 9.0069x; 9.0069x over previous
import functools
import math

import jax
import jax.numpy as jnp
from jax import lax
from jax.experimental import pallas as pl
from jax.experimental.pallas import tpu as pltpu
from jax.experimental.pallas import tpu_sc as plsc

N = 10000
B = 100
L = 100
D = 128
E = 320000
R = 4
NSTEP = 2
NR = N * R

NC = 2
NS = 16
NW = NC * NS
EPW = E // NW
CH = 80
NCH = EPW // CH
RPT = N // NS
CRP = NR // NS


def _enc_body(x_ref, w_ref, b_ref, o_ref):
    y = jnp.dot(x_ref[...], w_ref[...], preferred_element_type=jnp.float32)
    o_ref[...] = jnp.maximum(y + b_ref[...], 0.0)


def _enc(x, w_t, b):
    blk = 2000
    return pl.pallas_call(
        _enc_body,
        grid=(N // blk,),
        in_specs=[
            pl.BlockSpec((blk, D), lambda i: (i, 0)),
            pl.BlockSpec((D, D), lambda i: (0, 0)),
            pl.BlockSpec((1, D), lambda i: (0, 0)),
        ],
        out_specs=pl.BlockSpec((blk, D), lambda i: (i, 0)),
        out_shape=jax.ShapeDtypeStruct((N, D), jnp.float32),
    )(x, w_t, b.reshape(1, D))


def _inv_body(c0_ref, c1_ref, o_ref):
    s = c0_ref[:, 0:1] + c1_ref[:, 0:1]
    inv = 1.0 / jnp.maximum(s, 1.0)
    o_ref[...] = jnp.broadcast_to(inv, o_ref.shape)


def _prep_inv(cnt0, cnt1):
    blk = 4000
    return pl.pallas_call(
        _inv_body,
        grid=(NR // blk,),
        in_specs=[
            pl.BlockSpec((blk, 16), lambda i: (i, 0)),
            pl.BlockSpec((blk, 16), lambda i: (i, 0)),
        ],
        out_specs=pl.BlockSpec((blk, 2 * D), lambda i: (i, 0)),
        out_shape=jax.ShapeDtypeStruct((NR, 2 * D), jnp.float32),
    )(cnt0, cnt1)


def _pre_body(h_ref, w_ref, bias_ref, inv_ref, skip_ref, xl_ref, gb_ref):
    y = jnp.dot(h_ref[...], w_ref[...], preferred_element_type=jnp.float32)
    skip_ref[...] = jnp.maximum(y[:, 256:384] * y[:, 0:128] + y[:, 128:256], 0.0)
    xl_ref[...] = y[:, 384:896]
    gb_ref[...] = (y[:, 896:1920] + bias_ref[...]) * inv_ref[...]


def _pre(h, w_cat, bias_cat, inv_n):
    blk = 1000
    return pl.pallas_call(
        _pre_body,
        grid=(N // blk,),
        in_specs=[
            pl.BlockSpec((blk, D), lambda i: (i, 0)),
            pl.BlockSpec((D, 15 * D), lambda i: (0, 0)),
            pl.BlockSpec((1, 8 * D), lambda i: (0, 0)),
            pl.BlockSpec((blk, 8 * D), lambda i: (i, 0)),
        ],
        out_specs=[
            pl.BlockSpec((blk, D), lambda i: (i, 0)),
            pl.BlockSpec((blk, 4 * D), lambda i: (i, 0)),
            pl.BlockSpec((blk, 8 * D), lambda i: (i, 0)),
        ],
        out_shape=[
            jax.ShapeDtypeStruct((N, D), jnp.float32),
            jax.ShapeDtypeStruct((N, 4 * D), jnp.float32),
            jax.ShapeDtypeStruct((N, 8 * D), jnp.float32),
        ],
    )(h, w_cat, bias_cat, inv_n)


def _attn_body(h_ref, m_ref, q_ref):
    hm = h_ref[0] * m_ref[0]
    logits = lax.dot_general(
        hm, hm, (((1,), (1,)), ((), ())), preferred_element_type=jnp.float32
    ) * (1.0 / math.sqrt(D))
    mx = jnp.max(logits, axis=-1, keepdims=True)
    e = jnp.exp(logits - mx)
    aw = e / jnp.sum(e, axis=-1, keepdims=True)
    a = jnp.dot(aw, hm, preferred_element_type=jnp.float32)
    q_ref[0, 0] = jnp.mean(a, axis=0)


def _attn(h3, mask):
    out = pl.pallas_call(
        _attn_body,
        grid=(B,),
        in_specs=[
            pl.BlockSpec((1, L, D), lambda i: (i, 0, 0)),
            pl.BlockSpec((1, L, 1), lambda i: (i, 0, 0)),
        ],
        out_specs=pl.BlockSpec((1, 1, D), lambda i: (i, 0, 0)),
        out_shape=jax.ShapeDtypeStruct((B, 1, D), jnp.float32),
    )(h3, mask)
    return out.reshape(B, D)


def _gru_body(h_ref, skip_ref, p0_ref, p1_ref, sn_ref, wa_ref, wx_ref,
              ws_ref, wh1_ref, bias_ref, o_ref):
    h = h_ref[...]
    a = skip_ref[...] + p0_ref[...] + p1_ref[...]
    sn = sn_ref[...]
    ya = jnp.dot(a, wa_ref[...], preferred_element_type=jnp.float32)
    yx = jnp.dot(h, wx_ref[...], preferred_element_type=jnp.float32)
    ys = jnp.dot(sn, ws_ref[...], preferred_element_type=jnp.float32)
    bias = bias_ref[...]
    z = jax.nn.sigmoid(ya[:, 0:D] + yx[:, 0:D] + ys[:, 0:D] + bias[:, 0:D])
    r = jax.nn.sigmoid(
        ya[:, D:2 * D] + yx[:, D:2 * D] + ys[:, D:2 * D] + bias[:, D:2 * D])
    hr = jnp.dot(h * r, wh1_ref[...], preferred_element_type=jnp.float32)
    hh = jnp.maximum(
        ya[:, 2 * D:3 * D] + hr + ys[:, 2 * D:3 * D] + bias[:, 2 * D:3 * D], 0.0)
    o_ref[...] = hh * z + h * (1.0 - z)


def _gru(h, skip, p0, p1, sn, wa, wx, ws, wh1, bias):
    blk = 2000
    return pl.pallas_call(
        _gru_body,
        grid=(N // blk,),
        in_specs=[
            pl.BlockSpec((blk, D), lambda i: (i, 0)),
            pl.BlockSpec((blk, D), lambda i: (i, 0)),
            pl.BlockSpec((blk, D), lambda i: (i, 0)),
            pl.BlockSpec((blk, D), lambda i: (i, 0)),
            pl.BlockSpec((blk, D), lambda i: (i, 0)),
            pl.BlockSpec((D, 3 * D), lambda i: (0, 0)),
            pl.BlockSpec((D, 2 * D), lambda i: (0, 0)),
            pl.BlockSpec((D, 3 * D), lambda i: (0, 0)),
            pl.BlockSpec((D, D), lambda i: (0, 0)),
            pl.BlockSpec((1, 3 * D), lambda i: (0, 0)),
        ],
        out_specs=pl.BlockSpec((blk, D), lambda i: (i, 0)),
        out_shape=jax.ShapeDtypeStruct((N, D), jnp.float32),
    )(h, skip, p0, p1, sn, wa, wx, ws, wh1, bias)


def _count_sc(dstr):
    mesh = plsc.VectorSubcoreMesh(core_axis_name="c", subcore_axis_name="s")

    @functools.partial(
        pl.kernel,
        out_type=jax.ShapeDtypeStruct((NC, NR, 16), jnp.float32),
        mesh=mesh,
        compiler_params=pltpu.CompilerParams(use_tc_tiling_on_sc=False),
        scratch_types=[
            pltpu.VMEM((CH,), jnp.int32),
            pltpu.VMEM((CH, 16), jnp.float32),
            pltpu.VMEM((100, 16), jnp.float32),
            pltpu.VMEM_SHARED((NR, 16), jnp.float32),
        ],
    )
    def k(dstr_hbm, out_hbm, ibuf, ones_v, zbuf, cnt_sh):
        c = lax.axis_index("c")
        s = lax.axis_index("s")
        wid = s * NC + c
        lanes = lax.iota(jnp.int32, 16)
        unit = jnp.where(lanes == 0, 1.0, 0.0).astype(jnp.float32)
        zero = jnp.zeros((16,), jnp.float32)

        def fill(i, _):
            ones_v[i, :] = unit
            return 0

        lax.fori_loop(0, CH, fill, 0)

        def fillz(i, _):
            zbuf[i, :] = zero
            return 0

        lax.fori_loop(0, 100, fillz, 0)

        def zcp(i, _):
            pltpu.sync_copy(zbuf, cnt_sh.at[pl.ds(s * CRP + i * 100, 100)])
            return 0

        lax.fori_loop(0, CRP // 100, zcp, 0)
        plsc.subcore_barrier()

        def chunk(i, _):
            base = wid * EPW + i * CH
            pltpu.sync_copy(dstr_hbm.at[pl.ds(base, CH)], ibuf)
            pltpu.sync_copy(ones_v, cnt_sh.at[ibuf], add=True)
            return 0

        lax.fori_loop(0, NCH, chunk, 0)
        plsc.subcore_barrier()

        def dump(i, _):
            sl = pl.ds(s * CRP + i * 100, 100)
            pltpu.sync_copy(cnt_sh.at[sl], out_hbm.at[c, sl])
            return 0

        lax.fori_loop(0, CRP // 100, dump, 0)

    return k(dstr)


def _edges_sc(xl_tab, gb_tab, srcr, dstr):
    mesh = plsc.VectorSubcoreMesh(core_axis_name="c", subcore_axis_name="s")

    @functools.partial(
        pl.kernel,
        out_type=jax.ShapeDtypeStruct((NC, N, D), jnp.float32),
        mesh=mesh,
        compiler_params=pltpu.CompilerParams(use_tc_tiling_on_sc=False),
        scratch_types=[
            pltpu.VMEM((CH,), jnp.int32),
            pltpu.VMEM((CH,), jnp.int32),
            pltpu.VMEM((CH,), jnp.int32),
            pltpu.VMEM((CH, D), jnp.float32),
            pltpu.VMEM((CH, 2 * D), jnp.float32),
            pltpu.VMEM((CH, D), jnp.float32),
            pltpu.VMEM_SHARED((N, D), jnp.float32),
            pltpu.SemaphoreType.DMA,
            pltpu.SemaphoreType.DMA,
        ],
    )
    def k(xl_hbm, gb_hbm, srcr_hbm, dstr_hbm, out_hbm,
          isrc, idst, dstb, xbuf, gbbuf, msgbuf, accum, sem1, sem2):
        c = lax.axis_index("c")
        s = lax.axis_index("s")
        wid = s * NC + c
        ebase = wid * EPW
        zero = jnp.zeros((16,), jnp.float32)

        def zb(i, _):
            for j in range(D // 16):
                msgbuf[i, pl.ds(j * 16, 16)] = zero
            return 0

        lax.fori_loop(0, CH, zb, 0)
        rbase = s * RPT
        for i in range(RPT // CH):
            pltpu.sync_copy(msgbuf, accum.at[pl.ds(rbase + i * CH, CH)])
        rem = RPT - (RPT // CH) * CH
        pltpu.sync_copy(msgbuf.at[pl.ds(0, rem)],
                        accum.at[pl.ds(rbase + RPT - rem, rem)])
        plsc.subcore_barrier()

        def chunk(ci, _):
            base = ebase + ci * CH
            pltpu.sync_copy(srcr_hbm.at[pl.ds(base, CH)], isrc)
            pltpu.sync_copy(dstr_hbm.at[pl.ds(base, CH)], idst)
            g1 = pltpu.async_copy(xl_hbm.at[isrc], xbuf, sem1)
            g2 = pltpu.async_copy(gb_hbm.at[idst], gbbuf, sem2)
            for j in range(CH // 16):
                dstb[pl.ds(j * 16, 16)] = lax.shift_right_logical(
                    idst[pl.ds(j * 16, 16)], 2)
            g1.wait()
            g2.wait()

            def edge(e, _):
                for j in range(D // 16):
                    g = gbbuf[e, pl.ds(D + j * 16, 16)]
                    bt = gbbuf[e, pl.ds(j * 16, 16)]
                    xv = xbuf[e, pl.ds(j * 16, 16)]
                    msgbuf[e, pl.ds(j * 16, 16)] = jnp.maximum(g * xv + bt, 0.0)
                return 0

            lax.fori_loop(0, CH, edge, 0)
            pltpu.sync_copy(msgbuf, accum.at[dstb], add=True)
            return 0

        lax.fori_loop(0, NCH, chunk, 0)
        plsc.subcore_barrier()
        pltpu.sync_copy(accum.at[pl.ds(rbase, RPT)],
                        out_hbm.at[c, pl.ds(rbase, RPT)])

    return k(xl_tab, gb_tab, srcr, dstr)


def kernel(x, edge_index, edge_type, mask, params):
    src = edge_index[0].astype(jnp.int32)
    dst = edge_index[1].astype(jnp.int32)
    rt = edge_type.astype(jnp.int32)
    srcr = src * R + rt
    dstr = dst * R + rt

    cnt = _count_sc(dstr)
    inv_exp = _prep_inv(cnt[0], cnt[1])
    inv_n = inv_exp.reshape(N, 8 * D)

    p = params
    h = _enc(x, p["enc"]["W"].T, p["enc"]["b"])

    gp = p["gru_s"]
    wa = jnp.concatenate(
        [gp["z0"]["W"].T, gp["r0"]["W"].T, gp["h0"]["W"].T], axis=1)
    wx = jnp.concatenate([gp["z1"]["W"].T, gp["r1"]["W"].T], axis=1)
    ws = jnp.concatenate(
        [gp["z2"]["W"].T, gp["r2"]["W"].T, gp["h2"]["W"].T], axis=1)
    wh1 = gp["h1"]["W"].T
    gbias = jnp.concatenate([
        gp["z0"]["b"] + gp["z1"]["b"] + gp["z2"]["b"],
        gp["r0"]["b"] + gp["r1"]["b"] + gp["r2"]["b"],
        gp["h0"]["b"] + gp["h1"]["b"] + gp["h2"]["b"],
    ]).reshape(1, 3 * D)

    for step in range(NSTEP):
        cp = p["conv"][step]
        w_cat = jnp.concatenate(
            [cp["lin_skip"]["W"].T, cp["film_skip"]["W"].T]
            + [cp["lins"][r]["W"].T for r in range(R)]
            + [cp["films"][r]["W"].T for r in range(R)],
            axis=1)
        bias_cat = jnp.concatenate(
            [cp["films"][r]["b"] for r in range(R)]).reshape(1, 8 * D)

        skip, xl, gb = _pre(h, w_cat, bias_cat, inv_n)
        xl_tab = xl.reshape(NR, D)
        gb_tab = gb.reshape(NR, 2 * D)

        q = _attn(h.reshape(B, L, D), mask)
        sn = jnp.repeat(q, L, axis=0)

        parts = _edges_sc(xl_tab, gb_tab, srcr, dstr)
        h = _gru(h, skip, parts[0], parts[1], sn, wa, wx, ws, wh1, gbias)

    return h.reshape(B, L, D)


# speedup vs baseline: 22.4155x; 2.4887x over previous
import functools
import math

import jax
import jax.numpy as jnp
from jax import lax
from jax.experimental import pallas as pl
from jax.experimental.pallas import tpu as pltpu
from jax.experimental.pallas import tpu_sc as plsc

N = 10000
B = 100
L = 100
D = 128
E = 320000
R = 4
NSTEP = 2
NR = N * R

NC = 2
NS = 16
DH = D // NC
EPT = E // NS
CH = 80
NCHE = EPT // CH
NW = NC * NS
EPW = E // NW
NCHC = EPW // CH
RPT = N // NS
CRP = NR // NS


def _enc_body(x_ref, w_ref, b_ref, o_ref):
    y = jnp.dot(x_ref[...], w_ref[...], preferred_element_type=jnp.float32)
    o_ref[...] = jnp.maximum(y + b_ref[...], 0.0)


def _enc(x, w_t, b):
    blk = 2000
    return pl.pallas_call(
        _enc_body,
        grid=(N // blk,),
        in_specs=[
            pl.BlockSpec((blk, D), lambda i: (i, 0)),
            pl.BlockSpec((D, D), lambda i: (0, 0)),
            pl.BlockSpec((1, D), lambda i: (0, 0)),
        ],
        out_specs=pl.BlockSpec((blk, D), lambda i: (i, 0)),
        out_shape=jax.ShapeDtypeStruct((N, D), jnp.float32),
    )(x, w_t, b.reshape(1, D))


def _inv_body(c0_ref, c1_ref, o_ref):
    s = c0_ref[:, 0:1] + c1_ref[:, 0:1]
    inv = 1.0 / jnp.maximum(s, 1.0)
    o_ref[...] = jnp.broadcast_to(inv, o_ref.shape)


def _prep_inv(cnt0, cnt1):
    blk = 4000
    return pl.pallas_call(
        _inv_body,
        grid=(NR // blk,),
        in_specs=[
            pl.BlockSpec((blk, 16), lambda i: (i, 0)),
            pl.BlockSpec((blk, 16), lambda i: (i, 0)),
        ],
        out_specs=pl.BlockSpec((blk, D), lambda i: (i, 0)),
        out_shape=jax.ShapeDtypeStruct((NR, D), jnp.float32),
    )(cnt0, cnt1)


def _pre_body(h_ref, w_ref, bias_ref, inv_ref, skip_ref, xl_ref, gb_ref):
    y = jnp.dot(h_ref[...], w_ref[...], preferred_element_type=jnp.float32)
    skip_ref[...] = jnp.maximum(y[:, 256:384] * y[:, 0:128] + y[:, 128:256], 0.0)
    xl_ref[...] = y[:, 384:896]
    inv = inv_ref[...]
    bias = bias_ref[...]
    gb_ref[:, 0:512] = (y[:, 896:1408] + bias[:, 0:512]) * inv
    gb_ref[:, 512:1024] = (y[:, 1408:1920] + bias[:, 512:1024]) * inv


def _pre(h, w_cat, bias_cat, inv_n):
    blk = 1000
    return pl.pallas_call(
        _pre_body,
        grid=(N // blk,),
        in_specs=[
            pl.BlockSpec((blk, D), lambda i: (i, 0)),
            pl.BlockSpec((D, 15 * D), lambda i: (0, 0)),
            pl.BlockSpec((1, 8 * D), lambda i: (0, 0)),
            pl.BlockSpec((blk, 4 * D), lambda i: (i, 0)),
        ],
        out_specs=[
            pl.BlockSpec((blk, D), lambda i: (i, 0)),
            pl.BlockSpec((blk, 4 * D), lambda i: (i, 0)),
            pl.BlockSpec((blk, 8 * D), lambda i: (i, 0)),
        ],
        out_shape=[
            jax.ShapeDtypeStruct((N, D), jnp.float32),
            jax.ShapeDtypeStruct((N, 4 * D), jnp.float32),
            jax.ShapeDtypeStruct((N, 8 * D), jnp.float32),
        ],
    )(h, w_cat, bias_cat, inv_n)


def _attn_body(h_ref, m_ref, q_ref):
    hm = h_ref[0] * m_ref[0]
    logits = lax.dot_general(
        hm, hm, (((1,), (1,)), ((), ())), preferred_element_type=jnp.float32
    ) * (1.0 / math.sqrt(D))
    mx = jnp.max(logits, axis=-1, keepdims=True)
    e = jnp.exp(logits - mx)
    aw = e / jnp.sum(e, axis=-1, keepdims=True)
    a = jnp.dot(aw, hm, preferred_element_type=jnp.float32)
    q_ref[0, 0] = jnp.mean(a, axis=0)


def _attn(h3, mask):
    out = pl.pallas_call(
        _attn_body,
        grid=(B,),
        in_specs=[
            pl.BlockSpec((1, L, D), lambda i: (i, 0, 0)),
            pl.BlockSpec((1, L, 1), lambda i: (i, 0, 0)),
        ],
        out_specs=pl.BlockSpec((1, 1, D), lambda i: (i, 0, 0)),
        out_shape=jax.ShapeDtypeStruct((B, 1, D), jnp.float32),
    )(h3, mask)
    return out.reshape(B, D)


def _gru_body(h_ref, skip_ref, p_ref, sn_ref, wa_ref, wx_ref,
              ws_ref, wh1_ref, bias_ref, o_ref):
    h = h_ref[...]
    a = skip_ref[...] + p_ref[...]
    sn = sn_ref[...]
    ya = jnp.dot(a, wa_ref[...], preferred_element_type=jnp.float32)
    yx = jnp.dot(h, wx_ref[...], preferred_element_type=jnp.float32)
    ys = jnp.dot(sn, ws_ref[...], preferred_element_type=jnp.float32)
    bias = bias_ref[...]
    z = jax.nn.sigmoid(ya[:, 0:D] + yx[:, 0:D] + ys[:, 0:D] + bias[:, 0:D])
    r = jax.nn.sigmoid(
        ya[:, D:2 * D] + yx[:, D:2 * D] + ys[:, D:2 * D] + bias[:, D:2 * D])
    hr = jnp.dot(h * r, wh1_ref[...], preferred_element_type=jnp.float32)
    hh = jnp.maximum(
        ya[:, 2 * D:3 * D] + hr + ys[:, 2 * D:3 * D] + bias[:, 2 * D:3 * D], 0.0)
    o_ref[...] = hh * z + h * (1.0 - z)


def _gru(h, skip, p, sn, wa, wx, ws, wh1, bias):
    blk = 2000
    return pl.pallas_call(
        _gru_body,
        grid=(N // blk,),
        in_specs=[
            pl.BlockSpec((blk, D), lambda i: (i, 0)),
            pl.BlockSpec((blk, D), lambda i: (i, 0)),
            pl.BlockSpec((blk, D), lambda i: (i, 0)),
            pl.BlockSpec((blk, D), lambda i: (i, 0)),
            pl.BlockSpec((D, 3 * D), lambda i: (0, 0)),
            pl.BlockSpec((D, 2 * D), lambda i: (0, 0)),
            pl.BlockSpec((D, 3 * D), lambda i: (0, 0)),
            pl.BlockSpec((D, D), lambda i: (0, 0)),
            pl.BlockSpec((1, 3 * D), lambda i: (0, 0)),
        ],
        out_specs=pl.BlockSpec((blk, D), lambda i: (i, 0)),
        out_shape=jax.ShapeDtypeStruct((N, D), jnp.float32),
    )(h, skip, p, sn, wa, wx, ws, wh1, bias)


def _count_sc(dstr4):
    mesh = plsc.VectorSubcoreMesh(core_axis_name="c", subcore_axis_name="s")

    @functools.partial(
        pl.kernel,
        out_type=jax.ShapeDtypeStruct((NC, NR, 16), jnp.float32),
        mesh=mesh,
        compiler_params=pltpu.CompilerParams(use_tc_tiling_on_sc=False),
        scratch_types=[
            pltpu.VMEM((EPW,), jnp.int32),
            pltpu.VMEM((NCHC, CH), jnp.int32),
            pltpu.VMEM((CH, 16), jnp.float32),
            pltpu.VMEM((100, 16), jnp.float32),
            pltpu.VMEM_SHARED((NR, 16), jnp.float32),
            pltpu.SemaphoreType.DMA,
        ],
    )
    def k(dstr_hbm, out_hbm, dbuf, ibuf2d, ones_v, zbuf, cnt_sh, sem):
        c = lax.axis_index("c")
        s = lax.axis_index("s")
        wid = s * NC + c
        lanes = lax.iota(jnp.int32, 16)
        unit = jnp.where(lanes == 0, 1.0, 0.0).astype(jnp.float32)
        zero = jnp.zeros((16,), jnp.float32)

        pltpu.sync_copy(dstr_hbm.at[pl.ds(wid * EPW, EPW)], dbuf)

        def mkidx(ci, _):
            for j in range(CH // 16):
                ibuf2d[ci, pl.ds(16 * j, 16)] = dbuf[pl.ds(ci * CH + 16 * j, 16)]
            return 0

        lax.fori_loop(0, NCHC, mkidx, 0)

        def fill(i, _):
            ones_v[i, :] = unit
            return 0

        lax.fori_loop(0, CH, fill, 0)

        def fillz(i, _):
            zbuf[i, :] = zero
            return 0

        lax.fori_loop(0, 100, fillz, 0)

        def zcp(i, _):
            pltpu.sync_copy(zbuf, cnt_sh.at[pl.ds(s * CRP + i * 100, 100)])
            return 0

        lax.fori_loop(0, CRP // 100, zcp, 0)
        plsc.subcore_barrier()

        def chunk(i, _):
            pltpu.async_copy(ones_v, cnt_sh.at[ibuf2d.at[i]], sem, add=True)

            @pl.when(i >= 4)
            def _():
                pltpu.make_async_copy(
                    ones_v, cnt_sh.at[pl.ds(0, CH)], sem).wait()

            return 0

        lax.fori_loop(0, NCHC, chunk, 0)
        for _ in range(4):
            pltpu.make_async_copy(ones_v, cnt_sh.at[pl.ds(0, CH)], sem).wait()
        plsc.subcore_barrier()

        def dump(i, _):
            sl = pl.ds(s * CRP + i * 100, 100)
            pltpu.sync_copy(cnt_sh.at[sl], out_hbm.at[c, sl])
            return 0

        lax.fori_loop(0, CRP // 100, dump, 0)

    return k(dstr4)


def _edges_sc(xl_tab, gb_tab, srcr8, dstr8):
    mesh = plsc.VectorSubcoreMesh(core_axis_name="c", subcore_axis_name="s")

    @functools.partial(
        pl.kernel,
        out_type=jax.ShapeDtypeStruct((N, D), jnp.float32),
        mesh=mesh,
        compiler_params=pltpu.CompilerParams(use_tc_tiling_on_sc=False),
        scratch_types=[
            pltpu.VMEM((EPT,), jnp.int32),
            pltpu.VMEM((EPT,), jnp.int32),
            pltpu.VMEM((2, CH), jnp.int32),
            pltpu.VMEM((2, CH, DH), jnp.float32),
            pltpu.VMEM((2, CH, 2 * DH), jnp.float32),
            pltpu.VMEM((2, CH, DH), jnp.float32),
            pltpu.VMEM_SHARED((N, DH), jnp.float32),
            pltpu.SemaphoreType.DMA,
            pltpu.SemaphoreType.DMA,
            pltpu.SemaphoreType.DMA,
            pltpu.SemaphoreType.DMA,
            pltpu.SemaphoreType.DMA,
            pltpu.SemaphoreType.DMA,
        ],
    )
    def k(xl_hbm, gb_hbm, srcr_hbm, dstr_hbm, out_hbm,
          isrc, idst, dstb, xbuf, gbbuf, msgbuf, accum,
          gx0, gx1, gg0, gg1, ss0, ss1):
        gx = (gx0, gx1)
        gg = (gg0, gg1)
        ss = (ss0, ss1)
        c = lax.axis_index("c")
        s = lax.axis_index("s")
        ebase = s * EPT
        zero = jnp.zeros((16,), jnp.float32)

        pltpu.sync_copy(srcr_hbm.at[pl.ds(ebase, EPT)], isrc)
        pltpu.sync_copy(dstr_hbm.at[pl.ds(ebase, EPT)], idst)
        off = c * R

        def adj(i, _):
            sl = pl.ds(16 * i, 16)
            isrc[sl] = isrc[sl] + off
            idst[sl] = idst[sl] + off
            return 0

        lax.fori_loop(0, EPT // 16, adj, 0)

        def zb(i, _):
            for j in range(DH // 16):
                msgbuf[0, i, pl.ds(j * 16, 16)] = zero
            return 0

        lax.fori_loop(0, CH, zb, 0)
        rbase = s * RPT
        for i in range(RPT // CH):
            pltpu.sync_copy(msgbuf.at[0], accum.at[pl.ds(rbase + i * CH, CH)])
        rem = RPT - (RPT // CH) * CH
        pltpu.sync_copy(msgbuf.at[0, pl.ds(0, rem)],
                        accum.at[pl.ds(rbase + RPT - rem, rem)])
        plsc.subcore_barrier()

        def fire(ci, b):
            pltpu.async_copy(
                xl_hbm.at[isrc.at[pl.ds(ci * CH, CH)]], xbuf.at[b], gx[b])
            pltpu.async_copy(
                gb_hbm.at[idst.at[pl.ds(ci * CH, CH)]], gbbuf.at[b], gg[b])

        fire(0, 0)
        fire(1, 1)

        def body(i, _):
            for b in range(2):
                ci = 2 * i + b
                pltpu.make_async_copy(
                    xl_hbm.at[pl.ds(0, CH)], xbuf.at[b], gx[b]).wait()
                pltpu.make_async_copy(
                    gb_hbm.at[pl.ds(0, CH)], gbbuf.at[b], gg[b]).wait()

                for j in range(CH // 16):
                    dstb[b, pl.ds(16 * j, 16)] = lax.shift_right_logical(
                        idst[pl.ds(ci * CH + 16 * j, 16)], 3)

                @pl.when(ci >= 2)
                def _():
                    pltpu.make_async_copy(
                        msgbuf.at[b], accum.at[pl.ds(0, CH)], ss[b]).wait()

                @plsc.parallel_loop(0, CH, unroll=2)
                def _(e):
                    for j in range(DH // 16):
                        g = gbbuf[b, e, pl.ds(DH + j * 16, 16)]
                        bt = gbbuf[b, e, pl.ds(j * 16, 16)]
                        xv = xbuf[b, e, pl.ds(j * 16, 16)]
                        msgbuf[b, e, pl.ds(j * 16, 16)] = jnp.maximum(
                            g * xv + bt, 0.0)

                pltpu.async_copy(
                    msgbuf.at[b], accum.at[dstb.at[b]], ss[b], add=True)

                @pl.when(ci + 2 < NCHE)
                def _():
                    fire(ci + 2, b)

            return 0

        lax.fori_loop(0, NCHE // 2, body, 0)
        for b in range(2):
            pltpu.make_async_copy(
                msgbuf.at[b], accum.at[pl.ds(0, CH)], ss[b]).wait()
        plsc.subcore_barrier()
        pltpu.sync_copy(accum.at[pl.ds(rbase, RPT)],
                        out_hbm.at[pl.ds(rbase, RPT), pl.ds(c * DH, DH)])

    return k(xl_tab, gb_tab, srcr8, dstr8)


def kernel(x, edge_index, edge_type, mask, params):
    src = edge_index[0].astype(jnp.int32)
    dst = edge_index[1].astype(jnp.int32)
    rt = edge_type.astype(jnp.int32)
    srcr8 = src * (2 * R) + rt
    dstr8 = dst * (2 * R) + rt
    dstr4 = dst * R + rt

    cnt = _count_sc(dstr4)
    inv_exp = _prep_inv(cnt[0], cnt[1])
    inv_n = inv_exp.reshape(N, 4 * D)

    p = params
    h = _enc(x, p["enc"]["W"].T, p["enc"]["b"])

    gp = p["gru_s"]
    wa = jnp.concatenate(
        [gp["z0"]["W"].T, gp["r0"]["W"].T, gp["h0"]["W"].T], axis=1)
    wx = jnp.concatenate([gp["z1"]["W"].T, gp["r1"]["W"].T], axis=1)
    ws = jnp.concatenate(
        [gp["z2"]["W"].T, gp["r2"]["W"].T, gp["h2"]["W"].T], axis=1)
    wh1 = gp["h1"]["W"].T
    gbias = jnp.concatenate([
        gp["z0"]["b"] + gp["z1"]["b"] + gp["z2"]["b"],
        gp["r0"]["b"] + gp["r1"]["b"] + gp["r2"]["b"],
        gp["h0"]["b"] + gp["h1"]["b"] + gp["h2"]["b"],
    ]).reshape(1, 3 * D)

    for step in range(NSTEP):
        cp = p["conv"][step]
        xl_cols = [cp["lins"][r]["W"].T[:, c * DH:(c + 1) * DH]
                   for c in range(NC) for r in range(R)]
        gb_cols = []
        bias_cols = []
        for c in range(NC):
            for r in range(R):
                wt = cp["films"][r]["W"].T
                bb = cp["films"][r]["b"]
                gb_cols.append(wt[:, c * DH:(c + 1) * DH])
                gb_cols.append(wt[:, D + c * DH:D + (c + 1) * DH])
                bias_cols.append(bb[c * DH:(c + 1) * DH])
                bias_cols.append(bb[D + c * DH:D + (c + 1) * DH])
        w_cat = jnp.concatenate(
            [cp["lin_skip"]["W"].T, cp["film_skip"]["W"].T] + xl_cols + gb_cols,
            axis=1)
        bias_cat = jnp.concatenate(bias_cols).reshape(1, 8 * D)

        skip, xl, gb = _pre(h, w_cat, bias_cat, inv_n)
        xl_tab = xl.reshape(2 * NR, DH)
        gb_tab = gb.reshape(2 * NR, 2 * DH)

        q = _attn(h.reshape(B, L, D), mask)
        sn = jnp.repeat(q, L, axis=0)

        part = _edges_sc(xl_tab, gb_tab, srcr8, dstr8)
        h = _gru(h, skip, part, sn, wa, wx, ws, wh1, gbias)

    return h.reshape(B, L, D)


# speedup vs baseline: 25.0753x; 1.1187x over previous
import functools
import math

import jax
import jax.numpy as jnp
from jax import lax
from jax.experimental import pallas as pl
from jax.experimental.pallas import tpu as pltpu
from jax.experimental.pallas import tpu_sc as plsc

N = 10000
B = 100
L = 100
D = 128
E = 320000
R = 4
NSTEP = 2
NR = N * R

NC = 2
NS = 16
DH = D // NC
EPT = E // NS
CH = 80
NCHE = EPT // CH
NW = NC * NS
EPW = E // NW
NCHC = EPW // CH
RPT = N // NS
CRP = NR // NS


def _enc_body(x_ref, w_ref, b_ref, o_ref):
    y = jnp.dot(x_ref[...], w_ref[...], preferred_element_type=jnp.float32)
    o_ref[...] = jnp.maximum(y + b_ref[...], 0.0)


def _enc(x, w_t, b):
    blk = 2000
    return pl.pallas_call(
        _enc_body,
        grid=(N // blk,),
        in_specs=[
            pl.BlockSpec((blk, D), lambda i: (i, 0)),
            pl.BlockSpec((D, D), lambda i: (0, 0)),
            pl.BlockSpec((1, D), lambda i: (0, 0)),
        ],
        out_specs=pl.BlockSpec((blk, D), lambda i: (i, 0)),
        out_shape=jax.ShapeDtypeStruct((N, D), jnp.float32),
    )(x, w_t, b.reshape(1, D))


def _inv_body(c0_ref, c1_ref, o_ref):
    s = c0_ref[:, 0:1] + c1_ref[:, 0:1]
    inv = 1.0 / jnp.maximum(s, 1.0)
    o_ref[...] = jnp.broadcast_to(inv, o_ref.shape)


def _prep_inv(cnt0, cnt1):
    blk = 4000
    return pl.pallas_call(
        _inv_body,
        grid=(NR // blk,),
        in_specs=[
            pl.BlockSpec((blk, 16), lambda i: (i, 0)),
            pl.BlockSpec((blk, 16), lambda i: (i, 0)),
        ],
        out_specs=pl.BlockSpec((blk, D), lambda i: (i, 0)),
        out_shape=jax.ShapeDtypeStruct((NR, D), jnp.float32),
    )(cnt0, cnt1)


def _pre_body(h_ref, w_ref, bias_ref, inv_ref, skip_ref, xl_ref, gb_ref):
    y = jnp.dot(h_ref[...], w_ref[...], preferred_element_type=jnp.float32)
    skip_ref[...] = jnp.maximum(y[:, 256:384] * y[:, 0:128] + y[:, 128:256], 0.0)
    xl_ref[...] = y[:, 384:896].astype(jnp.bfloat16)
    inv = inv_ref[...]
    bias = bias_ref[...]
    gb_ref[:, 0:512] = ((y[:, 896:1408] + bias[:, 0:512]) * inv
                        ).astype(jnp.bfloat16)
    gb_ref[:, 512:1024] = ((y[:, 1408:1920] + bias[:, 512:1024]) * inv
                           ).astype(jnp.bfloat16)


def _pre(h, w_cat, bias_cat, inv_n):
    blk = 2000
    return pl.pallas_call(
        _pre_body,
        grid=(N // blk,),
        in_specs=[
            pl.BlockSpec((blk, D), lambda i: (i, 0)),
            pl.BlockSpec((D, 15 * D), lambda i: (0, 0)),
            pl.BlockSpec((1, 8 * D), lambda i: (0, 0)),
            pl.BlockSpec((blk, 4 * D), lambda i: (i, 0)),
        ],
        out_specs=[
            pl.BlockSpec((blk, D), lambda i: (i, 0)),
            pl.BlockSpec((blk, 4 * D), lambda i: (i, 0)),
            pl.BlockSpec((blk, 8 * D), lambda i: (i, 0)),
        ],
        out_shape=[
            jax.ShapeDtypeStruct((N, D), jnp.float32),
            jax.ShapeDtypeStruct((N, 4 * D), jnp.bfloat16),
            jax.ShapeDtypeStruct((N, 8 * D), jnp.bfloat16),
        ],
    )(h, w_cat, bias_cat, inv_n)


def _attn_body(h_ref, m_ref, q_ref):
    hm = h_ref[0] * m_ref[0]
    logits = lax.dot_general(
        hm, hm, (((1,), (1,)), ((), ())), preferred_element_type=jnp.float32
    ) * (1.0 / math.sqrt(D))
    mx = jnp.max(logits, axis=-1, keepdims=True)
    e = jnp.exp(logits - mx)
    aw = e / jnp.sum(e, axis=-1, keepdims=True)
    a = jnp.dot(aw, hm, preferred_element_type=jnp.float32)
    q_ref[0, 0] = jnp.mean(a, axis=0)


def _attn(h3, mask):
    out = pl.pallas_call(
        _attn_body,
        grid=(B,),
        in_specs=[
            pl.BlockSpec((1, L, D), lambda i: (i, 0, 0)),
            pl.BlockSpec((1, L, 1), lambda i: (i, 0, 0)),
        ],
        out_specs=pl.BlockSpec((1, 1, D), lambda i: (i, 0, 0)),
        out_shape=jax.ShapeDtypeStruct((B, 1, D), jnp.float32),
    )(h3, mask)
    return out.reshape(B, D)


def _gru_body(h_ref, skip_ref, p_ref, sn_ref, wa_ref, wx_ref,
              ws_ref, wh1_ref, bias_ref, o_ref):
    h = h_ref[...]
    a = skip_ref[...] + p_ref[...]
    sn = sn_ref[...]
    ya = jnp.dot(a, wa_ref[...], preferred_element_type=jnp.float32)
    yx = jnp.dot(h, wx_ref[...], preferred_element_type=jnp.float32)
    ys = jnp.dot(sn, ws_ref[...], preferred_element_type=jnp.float32)
    bias = bias_ref[...]
    z = jax.nn.sigmoid(ya[:, 0:D] + yx[:, 0:D] + ys[:, 0:D] + bias[:, 0:D])
    r = jax.nn.sigmoid(
        ya[:, D:2 * D] + yx[:, D:2 * D] + ys[:, D:2 * D] + bias[:, D:2 * D])
    hr = jnp.dot(h * r, wh1_ref[...], preferred_element_type=jnp.float32)
    hh = jnp.maximum(
        ya[:, 2 * D:3 * D] + hr + ys[:, 2 * D:3 * D] + bias[:, 2 * D:3 * D], 0.0)
    o_ref[...] = hh * z + h * (1.0 - z)


def _gru(h, skip, p, sn, wa, wx, ws, wh1, bias):
    blk = 2000
    return pl.pallas_call(
        _gru_body,
        grid=(N // blk,),
        in_specs=[
            pl.BlockSpec((blk, D), lambda i: (i, 0)),
            pl.BlockSpec((blk, D), lambda i: (i, 0)),
            pl.BlockSpec((blk, D), lambda i: (i, 0)),
            pl.BlockSpec((blk, D), lambda i: (i, 0)),
            pl.BlockSpec((D, 3 * D), lambda i: (0, 0)),
            pl.BlockSpec((D, 2 * D), lambda i: (0, 0)),
            pl.BlockSpec((D, 3 * D), lambda i: (0, 0)),
            pl.BlockSpec((D, D), lambda i: (0, 0)),
            pl.BlockSpec((1, 3 * D), lambda i: (0, 0)),
        ],
        out_specs=pl.BlockSpec((blk, D), lambda i: (i, 0)),
        out_shape=jax.ShapeDtypeStruct((N, D), jnp.float32),
    )(h, skip, p, sn, wa, wx, ws, wh1, bias)


def _count_sc(dstr4):
    mesh = plsc.VectorSubcoreMesh(core_axis_name="c", subcore_axis_name="s")

    @functools.partial(
        pl.kernel,
        out_type=jax.ShapeDtypeStruct((NC, NR, 16), jnp.float32),
        mesh=mesh,
        compiler_params=pltpu.CompilerParams(use_tc_tiling_on_sc=False),
        scratch_types=[
            pltpu.VMEM((EPW,), jnp.int32),
            pltpu.VMEM((NCHC, CH), jnp.int32),
            pltpu.VMEM((CH, 16), jnp.float32),
            pltpu.VMEM((100, 16), jnp.float32),
            pltpu.VMEM_SHARED((NR, 16), jnp.float32),
            pltpu.SemaphoreType.DMA,
        ],
    )
    def k(dstr_hbm, out_hbm, dbuf, ibuf2d, ones_v, zbuf, cnt_sh, sem):
        c = lax.axis_index("c")
        s = lax.axis_index("s")
        wid = s * NC + c
        lanes = lax.iota(jnp.int32, 16)
        unit = jnp.where(lanes == 0, 1.0, 0.0).astype(jnp.float32)
        zero = jnp.zeros((16,), jnp.float32)

        pltpu.sync_copy(dstr_hbm.at[pl.ds(wid * EPW, EPW)], dbuf)

        def mkidx(ci, _):
            for j in range(CH // 16):
                ibuf2d[ci, pl.ds(16 * j, 16)] = dbuf[pl.ds(ci * CH + 16 * j, 16)]
            return 0

        lax.fori_loop(0, NCHC, mkidx, 0)

        def fill(i, _):
            ones_v[i, :] = unit
            return 0

        lax.fori_loop(0, CH, fill, 0)

        def fillz(i, _):
            zbuf[i, :] = zero
            return 0

        lax.fori_loop(0, 100, fillz, 0)

        def zcp(i, _):
            pltpu.sync_copy(zbuf, cnt_sh.at[pl.ds(s * CRP + i * 100, 100)])
            return 0

        lax.fori_loop(0, CRP // 100, zcp, 0)
        plsc.subcore_barrier()

        def chunk(i, _):
            pltpu.async_copy(ones_v, cnt_sh.at[ibuf2d.at[i]], sem, add=True)

            @pl.when(i >= 4)
            def _():
                pltpu.make_async_copy(
                    ones_v, cnt_sh.at[pl.ds(0, CH)], sem).wait()

            return 0

        lax.fori_loop(0, NCHC, chunk, 0)
        for _ in range(4):
            pltpu.make_async_copy(ones_v, cnt_sh.at[pl.ds(0, CH)], sem).wait()
        plsc.subcore_barrier()

        def dump(i, _):
            sl = pl.ds(s * CRP + i * 100, 100)
            pltpu.sync_copy(cnt_sh.at[sl], out_hbm.at[c, sl])
            return 0

        lax.fori_loop(0, CRP // 100, dump, 0)

    return k(dstr4)


def _edges_sc(xl_tab, gb_tab, srcr8, dstr8):
    mesh = plsc.VectorSubcoreMesh(core_axis_name="c", subcore_axis_name="s")

    @functools.partial(
        pl.kernel,
        out_type=jax.ShapeDtypeStruct((N, D), jnp.float32),
        mesh=mesh,
        compiler_params=pltpu.CompilerParams(
            use_tc_tiling_on_sc=False, needs_layout_passes=False),
        scratch_types=[
            pltpu.VMEM((EPT,), jnp.int32),
            pltpu.VMEM((EPT,), jnp.int32),
            pltpu.VMEM((2, CH), jnp.int32),
            pltpu.VMEM((2, CH, DH), jnp.bfloat16),
            pltpu.VMEM((2, CH, 2 * DH), jnp.bfloat16),
            pltpu.VMEM((2, CH, DH), jnp.float32),
            pltpu.VMEM_SHARED((N, DH), jnp.float32),
            pltpu.SemaphoreType.DMA,
            pltpu.SemaphoreType.DMA,
            pltpu.SemaphoreType.DMA,
            pltpu.SemaphoreType.DMA,
            pltpu.SemaphoreType.DMA,
            pltpu.SemaphoreType.DMA,
        ],
    )
    def k(xl_hbm, gb_hbm, srcr_hbm, dstr_hbm, out_hbm,
          isrc, idst, dstb, xbuf, gbbuf, msgbuf, accum,
          gx0, gx1, gg0, gg1, ss0, ss1):
        gx = (gx0, gx1)
        gg = (gg0, gg1)
        ss = (ss0, ss1)
        c = lax.axis_index("c")
        s = lax.axis_index("s")
        ebase = s * EPT
        zero = jnp.zeros((16,), jnp.float32)

        pltpu.sync_copy(srcr_hbm.at[pl.ds(ebase, EPT)], isrc)
        pltpu.sync_copy(dstr_hbm.at[pl.ds(ebase, EPT)], idst)
        off = c * R

        def adj(i, _):
            sl = pl.ds(16 * i, 16)
            isrc[sl] = isrc[sl] + off
            idst[sl] = idst[sl] + off
            return 0

        lax.fori_loop(0, EPT // 16, adj, 0)

        def zb(i, _):
            for j in range(DH // 16):
                msgbuf[0, i, pl.ds(j * 16, 16)] = zero
            return 0

        lax.fori_loop(0, CH, zb, 0)
        rbase = s * RPT
        for i in range(RPT // CH):
            pltpu.sync_copy(msgbuf.at[0], accum.at[pl.ds(rbase + i * CH, CH)])
        rem = RPT - (RPT // CH) * CH
        pltpu.sync_copy(msgbuf.at[0, pl.ds(0, rem)],
                        accum.at[pl.ds(rbase + RPT - rem, rem)])
        plsc.subcore_barrier()

        def fire(ci, b):
            pltpu.async_copy(
                xl_hbm.at[isrc.at[pl.ds(ci * CH, CH)]], xbuf.at[b], gx[b])
            pltpu.async_copy(
                gb_hbm.at[idst.at[pl.ds(ci * CH, CH)]], gbbuf.at[b], gg[b])

        fire(0, 0)
        fire(1, 1)

        def body(i, _):
            for b in range(2):
                ci = 2 * i + b
                pltpu.make_async_copy(
                    xl_hbm.at[pl.ds(0, CH)], xbuf.at[b], gx[b]).wait()
                pltpu.make_async_copy(
                    gb_hbm.at[pl.ds(0, CH)], gbbuf.at[b], gg[b]).wait()

                for j in range(CH // 16):
                    dstb[b, pl.ds(16 * j, 16)] = lax.shift_right_logical(
                        idst[pl.ds(ci * CH + 16 * j, 16)], 3)

                @pl.when(ci >= 2)
                def _():
                    pltpu.make_async_copy(
                        msgbuf.at[b], accum.at[pl.ds(0, CH)], ss[b]).wait()

                @plsc.parallel_loop(0, CH, unroll=2)
                def _(e):
                    for j in range(DH // 32):
                        gv = gbbuf[b, e, pl.ds(DH + 32 * j, 32)]
                        bv = gbbuf[b, e, pl.ds(32 * j, 32)]
                        xv = xbuf[b, e, pl.ds(32 * j, 32)]
                        g0, g1 = plsc.unpack(
                            gv, format=plsc.PackFormat.INTERLEAVED,
                            preferred_element_type=jnp.float32)
                        b0, b1 = plsc.unpack(
                            bv, format=plsc.PackFormat.INTERLEAVED,
                            preferred_element_type=jnp.float32)
                        x0, x1 = plsc.unpack(
                            xv, format=plsc.PackFormat.INTERLEAVED,
                            preferred_element_type=jnp.float32)
                        msgbuf[b, e, pl.ds(32 * j, 16)] = jnp.maximum(
                            g0 * x0 + b0, 0.0)
                        msgbuf[b, e, pl.ds(32 * j + 16, 16)] = jnp.maximum(
                            g1 * x1 + b1, 0.0)

                pltpu.async_copy(
                    msgbuf.at[b], accum.at[dstb.at[b]], ss[b], add=True)

                @pl.when(ci + 2 < NCHE)
                def _():
                    fire(ci + 2, b)

            return 0

        lax.fori_loop(0, NCHE // 2, body, 0)
        for b in range(2):
            pltpu.make_async_copy(
                msgbuf.at[b], accum.at[pl.ds(0, CH)], ss[b]).wait()
        plsc.subcore_barrier()
        pltpu.sync_copy(accum.at[pl.ds(rbase, RPT)],
                        out_hbm.at[pl.ds(rbase, RPT), pl.ds(c * DH, DH)])

    return k(xl_tab, gb_tab, srcr8, dstr8)


def _ilv(w):
    d = w.shape[-1]
    w2 = w.reshape(w.shape[:-1] + (d // 32, 2, 16))
    return jnp.swapaxes(w2, -1, -2).reshape(w.shape)


def kernel(x, edge_index, edge_type, mask, params):
    src = edge_index[0].astype(jnp.int32)
    dst = edge_index[1].astype(jnp.int32)
    rt = edge_type.astype(jnp.int32)
    srcr8 = src * (2 * R) + rt
    dstr8 = dst * (2 * R) + rt
    dstr4 = dst * R + rt

    cnt = _count_sc(dstr4)
    inv_exp = _prep_inv(cnt[0], cnt[1])
    inv_n = inv_exp.reshape(N, 4 * D)

    p = params
    h = _enc(x, p["enc"]["W"].T, p["enc"]["b"])

    gp = p["gru_s"]
    wa = jnp.concatenate(
        [gp["z0"]["W"].T, gp["r0"]["W"].T, gp["h0"]["W"].T], axis=1)
    wx = jnp.concatenate([gp["z1"]["W"].T, gp["r1"]["W"].T], axis=1)
    ws = jnp.concatenate(
        [gp["z2"]["W"].T, gp["r2"]["W"].T, gp["h2"]["W"].T], axis=1)
    wh1 = gp["h1"]["W"].T
    gbias = jnp.concatenate([
        gp["z0"]["b"] + gp["z1"]["b"] + gp["z2"]["b"],
        gp["r0"]["b"] + gp["r1"]["b"] + gp["r2"]["b"],
        gp["h0"]["b"] + gp["h1"]["b"] + gp["h2"]["b"],
    ]).reshape(1, 3 * D)

    for step in range(NSTEP):
        cp = p["conv"][step]
        xl_cols = [_ilv(cp["lins"][r]["W"].T[:, c * DH:(c + 1) * DH])
                   for c in range(NC) for r in range(R)]
        gb_cols = []
        bias_cols = []
        for c in range(NC):
            for r in range(R):
                wt = cp["films"][r]["W"].T
                bb = cp["films"][r]["b"]
                gb_cols.append(_ilv(wt[:, c * DH:(c + 1) * DH]))
                gb_cols.append(_ilv(wt[:, D + c * DH:D + (c + 1) * DH]))
                bias_cols.append(_ilv(bb[c * DH:(c + 1) * DH]))
                bias_cols.append(_ilv(bb[D + c * DH:D + (c + 1) * DH]))
        w_cat = jnp.concatenate(
            [cp["lin_skip"]["W"].T, cp["film_skip"]["W"].T] + xl_cols + gb_cols,
            axis=1)
        bias_cat = jnp.concatenate(bias_cols).reshape(1, 8 * D)

        skip, xl, gb = _pre(h, w_cat, bias_cat, inv_n)
        xl_tab = xl.reshape(2 * NR, DH)
        gb_tab = gb.reshape(2 * NR, 2 * DH)

        q = _attn(h.reshape(B, L, D), mask)
        sn = jnp.repeat(q, L, axis=0)

        part = _edges_sc(xl_tab, gb_tab, srcr8, dstr8)
        h = _gru(h, skip, part, sn, wa, wx, ws, wh1, gbias)

    return h.reshape(B, L, D)


# speedup vs baseline: 32.4396x; 1.2937x over previous
import functools
import math

import jax
import jax.numpy as jnp
from jax import lax
from jax.experimental import pallas as pl
from jax.experimental.pallas import tpu as pltpu
from jax.experimental.pallas import tpu_sc as plsc

N = 10000
B = 100
L = 100
D = 128
E = 320000
R = 4
NSTEP = 2
NR = N * R

NC = 2
NS = 16
DH = D // NC
EPT = E // NS
CH = 80
NCHE = EPT // CH
NW = NC * NS
EPW = E // NW
NCHC = EPW // CH
RPT = N // NS
CRP = NR // NS


def _enc_body(x_ref, w_ref, b_ref, o_ref):
    y = jnp.dot(x_ref[...], w_ref[...], preferred_element_type=jnp.float32)
    o_ref[...] = jnp.maximum(y + b_ref[...], 0.0)


def _enc(x, w_t, b):
    blk = 2000
    return pl.pallas_call(
        _enc_body,
        grid=(N // blk,),
        in_specs=[
            pl.BlockSpec((blk, D), lambda i: (i, 0)),
            pl.BlockSpec((D, D), lambda i: (0, 0)),
            pl.BlockSpec((1, D), lambda i: (0, 0)),
        ],
        out_specs=pl.BlockSpec((blk, D), lambda i: (i, 0)),
        out_shape=jax.ShapeDtypeStruct((N, D), jnp.float32),
    )(x, w_t, b.reshape(1, D))


def _inv_body(c0_ref, c1_ref, o_ref):
    s = c0_ref[:, 0:1] + c1_ref[:, 0:1]
    inv = 1.0 / jnp.maximum(s, 1.0)
    o_ref[...] = jnp.broadcast_to(inv, o_ref.shape)


def _prep_inv(cnt0, cnt1):
    blk = 4000
    return pl.pallas_call(
        _inv_body,
        grid=(NR // blk,),
        in_specs=[
            pl.BlockSpec((blk, 16), lambda i: (i, 0)),
            pl.BlockSpec((blk, 16), lambda i: (i, 0)),
        ],
        out_specs=pl.BlockSpec((blk, DH), lambda i: (i, 0)),
        out_shape=jax.ShapeDtypeStruct((NR, DH), jnp.float32),
    )(cnt0, cnt1)


def _pack_pair(lo, hi):
    lo_b = lax.bitcast_convert_type(lo.astype(jnp.bfloat16), jnp.uint16)
    hi_b = lax.bitcast_convert_type(hi.astype(jnp.bfloat16), jnp.uint16)
    u = lo_b.astype(jnp.uint32) | (hi_b.astype(jnp.uint32) << 16)
    return lax.bitcast_convert_type(u, jnp.float32)


def _pre_body(h_ref, w_ref, bias_ref, inv_ref, skip_ref, xl_ref, gb_ref):
    y = jnp.dot(h_ref[...], w_ref[...], preferred_element_type=jnp.float32)
    skip_ref[...] = jnp.maximum(y[:, 256:384] * y[:, 0:128] + y[:, 128:256], 0.0)
    xlp = _pack_pair(y[:, 384:640], y[:, 640:896])
    xl_ref[:, 0, :] = xlp[:, 0:128]
    xl_ref[:, 1, :] = xlp[:, 128:256]
    inv = inv_ref[...]
    bias = bias_ref[...]
    glo = (y[:, 896:1408] + bias[:, 0:512]) * inv
    ghi = (y[:, 1408:1920] + bias[:, 512:1024]) * inv
    gbp = _pack_pair(glo, ghi)
    gb_ref[:, 0, :] = gbp[:, 0:128]
    gb_ref[:, 1, :] = gbp[:, 128:256]
    gb_ref[:, 2, :] = gbp[:, 256:384]
    gb_ref[:, 3, :] = gbp[:, 384:512]


def _pre(h, w_cat, bias_cat, inv_n):
    blk = 2000
    return pl.pallas_call(
        _pre_body,
        grid=(N // blk,),
        in_specs=[
            pl.BlockSpec((blk, D), lambda i: (i, 0)),
            pl.BlockSpec((D, 15 * D), lambda i: (0, 0)),
            pl.BlockSpec((1, 8 * D), lambda i: (0, 0)),
            pl.BlockSpec((blk, 4 * D), lambda i: (i, 0)),
        ],
        out_specs=[
            pl.BlockSpec((blk, D), lambda i: (i, 0)),
            pl.BlockSpec((blk, 2, D), lambda i: (i, 0, 0)),
            pl.BlockSpec((blk, 4, D), lambda i: (i, 0, 0)),
        ],
        out_shape=[
            jax.ShapeDtypeStruct((N, D), jnp.float32),
            jax.ShapeDtypeStruct((N, 2, D), jnp.float32),
            jax.ShapeDtypeStruct((N, 4, D), jnp.float32),
        ],
    )(h, w_cat, bias_cat, inv_n)


def _attn_body(h_ref, m_ref, q_ref):
    hm = h_ref[0] * m_ref[0]
    logits = lax.dot_general(
        hm, hm, (((1,), (1,)), ((), ())), preferred_element_type=jnp.float32
    ) * (1.0 / math.sqrt(D))
    mx = jnp.max(logits, axis=-1, keepdims=True)
    e = jnp.exp(logits - mx)
    aw = e / jnp.sum(e, axis=-1, keepdims=True)
    a = jnp.dot(aw, hm, preferred_element_type=jnp.float32)
    q_ref[0, 0] = jnp.mean(a, axis=0)


def _attn(h3, mask):
    out = pl.pallas_call(
        _attn_body,
        grid=(B,),
        in_specs=[
            pl.BlockSpec((1, L, D), lambda i: (i, 0, 0)),
            pl.BlockSpec((1, L, 1), lambda i: (i, 0, 0)),
        ],
        out_specs=pl.BlockSpec((1, 1, D), lambda i: (i, 0, 0)),
        out_shape=jax.ShapeDtypeStruct((B, 1, D), jnp.float32),
    )(h3, mask)
    return out.reshape(B, D)


def _gru_body(h_ref, skip_ref, p_ref, sn_ref, wa_ref, wx_ref,
              ws_ref, wh1_ref, bias_ref, o_ref):
    h = h_ref[...]
    a = skip_ref[...] + p_ref[...]
    sn = sn_ref[...]
    ya = jnp.dot(a, wa_ref[...], preferred_element_type=jnp.float32)
    yx = jnp.dot(h, wx_ref[...], preferred_element_type=jnp.float32)
    ys = jnp.dot(sn, ws_ref[...], preferred_element_type=jnp.float32)
    bias = bias_ref[...]
    z = jax.nn.sigmoid(ya[:, 0:D] + yx[:, 0:D] + ys[:, 0:D] + bias[:, 0:D])
    r = jax.nn.sigmoid(
        ya[:, D:2 * D] + yx[:, D:2 * D] + ys[:, D:2 * D] + bias[:, D:2 * D])
    hr = jnp.dot(h * r, wh1_ref[...], preferred_element_type=jnp.float32)
    hh = jnp.maximum(
        ya[:, 2 * D:3 * D] + hr + ys[:, 2 * D:3 * D] + bias[:, 2 * D:3 * D], 0.0)
    o_ref[...] = hh * z + h * (1.0 - z)


def _gru(h, skip, p, sn, wa, wx, ws, wh1, bias):
    blk = 2000
    return pl.pallas_call(
        _gru_body,
        grid=(N // blk,),
        in_specs=[
            pl.BlockSpec((blk, D), lambda i: (i, 0)),
            pl.BlockSpec((blk, D), lambda i: (i, 0)),
            pl.BlockSpec((blk, D), lambda i: (i, 0)),
            pl.BlockSpec((blk, D), lambda i: (i, 0)),
            pl.BlockSpec((D, 3 * D), lambda i: (0, 0)),
            pl.BlockSpec((D, 2 * D), lambda i: (0, 0)),
            pl.BlockSpec((D, 3 * D), lambda i: (0, 0)),
            pl.BlockSpec((D, D), lambda i: (0, 0)),
            pl.BlockSpec((1, 3 * D), lambda i: (0, 0)),
        ],
        out_specs=pl.BlockSpec((blk, D), lambda i: (i, 0)),
        out_shape=jax.ShapeDtypeStruct((N, D), jnp.float32),
    )(h, skip, p, sn, wa, wx, ws, wh1, bias)


def _count_sc(dstr4):
    mesh = plsc.VectorSubcoreMesh(core_axis_name="c", subcore_axis_name="s")

    @functools.partial(
        pl.kernel,
        out_type=jax.ShapeDtypeStruct((NC, NR, 16), jnp.float32),
        mesh=mesh,
        compiler_params=pltpu.CompilerParams(use_tc_tiling_on_sc=False),
        scratch_types=[
            pltpu.VMEM((EPW,), jnp.int32),
            pltpu.VMEM((NCHC, CH), jnp.int32),
            pltpu.VMEM((CH, 16), jnp.float32),
            pltpu.VMEM((100, 16), jnp.float32),
            pltpu.VMEM_SHARED((NR, 16), jnp.float32),
            pltpu.SemaphoreType.DMA,
        ],
    )
    def k(dstr_hbm, out_hbm, dbuf, ibuf2d, ones_v, zbuf, cnt_sh, sem):
        c = lax.axis_index("c")
        s = lax.axis_index("s")
        wid = s * NC + c
        lanes = lax.iota(jnp.int32, 16)
        unit = jnp.where(lanes == 0, 1.0, 0.0).astype(jnp.float32)
        zero = jnp.zeros((16,), jnp.float32)

        pltpu.sync_copy(dstr_hbm.at[pl.ds(wid * EPW, EPW)], dbuf)

        def mkidx(ci, _):
            for j in range(CH // 16):
                ibuf2d[ci, pl.ds(16 * j, 16)] = dbuf[pl.ds(ci * CH + 16 * j, 16)]
            return 0

        lax.fori_loop(0, NCHC, mkidx, 0)

        def fill(i, _):
            ones_v[i, :] = unit
            return 0

        lax.fori_loop(0, CH, fill, 0)

        def fillz(i, _):
            zbuf[i, :] = zero
            return 0

        lax.fori_loop(0, 100, fillz, 0)

        def zcp(i, _):
            pltpu.sync_copy(zbuf, cnt_sh.at[pl.ds(s * CRP + i * 100, 100)])
            return 0

        lax.fori_loop(0, CRP // 100, zcp, 0)
        plsc.subcore_barrier()

        def chunk(i, _):
            pltpu.async_copy(ones_v, cnt_sh.at[ibuf2d.at[i]], sem, add=True)

            @pl.when(i >= 4)
            def _():
                pltpu.make_async_copy(
                    ones_v, cnt_sh.at[pl.ds(0, CH)], sem).wait()

            return 0

        lax.fori_loop(0, NCHC, chunk, 0)
        for _ in range(4):
            pltpu.make_async_copy(ones_v, cnt_sh.at[pl.ds(0, CH)], sem).wait()
        plsc.subcore_barrier()

        def dump(i, _):
            sl = pl.ds(s * CRP + i * 100, 100)
            pltpu.sync_copy(cnt_sh.at[sl], out_hbm.at[c, sl])
            return 0

        lax.fori_loop(0, CRP // 100, dump, 0)

    return k(dstr4)


def _edges_sc(xl_tab, gb_tab, srcr8, dstr8):
    mesh = plsc.VectorSubcoreMesh(core_axis_name="c", subcore_axis_name="s")

    @functools.partial(
        pl.kernel,
        out_type=jax.ShapeDtypeStruct((N, D), jnp.float32),
        mesh=mesh,
        compiler_params=pltpu.CompilerParams(
            use_tc_tiling_on_sc=False, needs_layout_passes=False),
        scratch_types=[
            pltpu.VMEM((EPT,), jnp.int32),
            pltpu.VMEM((EPT,), jnp.int32),
            pltpu.VMEM((2, CH), jnp.int32),
            pltpu.VMEM((2, CH, DH // 2), jnp.float32),
            pltpu.VMEM((2, CH, DH), jnp.float32),
            pltpu.VMEM((2, CH, DH), jnp.float32),
            pltpu.VMEM_SHARED((N, DH), jnp.float32),
            pltpu.SemaphoreType.DMA,
            pltpu.SemaphoreType.DMA,
            pltpu.SemaphoreType.DMA,
            pltpu.SemaphoreType.DMA,
            pltpu.SemaphoreType.DMA,
            pltpu.SemaphoreType.DMA,
        ],
    )
    def k(xl_hbm, gb_hbm, srcr_hbm, dstr_hbm, out_hbm,
          isrc, idst, dstb, xbuf, gbbuf, msgbuf, accum,
          gx0, gx1, gg0, gg1, ss0, ss1):
        gx = (gx0, gx1)
        gg = (gg0, gg1)
        ss = (ss0, ss1)
        c = lax.axis_index("c")
        s = lax.axis_index("s")
        ebase = s * EPT
        zero = jnp.zeros((16,), jnp.float32)

        pltpu.sync_copy(srcr_hbm.at[pl.ds(ebase, EPT)], isrc)
        pltpu.sync_copy(dstr_hbm.at[pl.ds(ebase, EPT)], idst)
        off = c * R

        def adj(i, _):
            sl = pl.ds(16 * i, 16)
            isrc[sl] = isrc[sl] + off
            idst[sl] = idst[sl] + off
            return 0

        lax.fori_loop(0, EPT // 16, adj, 0)

        def zb(i, _):
            for j in range(DH // 16):
                msgbuf[0, i, pl.ds(j * 16, 16)] = zero
            return 0

        lax.fori_loop(0, CH, zb, 0)
        rbase = s * RPT
        for i in range(RPT // CH):
            pltpu.sync_copy(msgbuf.at[0], accum.at[pl.ds(rbase + i * CH, CH)])
        rem = RPT - (RPT // CH) * CH
        pltpu.sync_copy(msgbuf.at[0, pl.ds(0, rem)],
                        accum.at[pl.ds(rbase + RPT - rem, rem)])
        plsc.subcore_barrier()

        def fire(ci, b):
            pltpu.async_copy(
                xl_hbm.at[isrc.at[pl.ds(ci * CH, CH)]], xbuf.at[b], gx[b])
            pltpu.async_copy(
                gb_hbm.at[idst.at[pl.ds(ci * CH, CH)]], gbbuf.at[b], gg[b])

        fire(0, 0)
        fire(1, 1)

        def body(i, _):
            for b in range(2):
                ci = 2 * i + b
                pltpu.make_async_copy(
                    xl_hbm.at[pl.ds(0, CH)], xbuf.at[b], gx[b]).wait()
                pltpu.make_async_copy(
                    gb_hbm.at[pl.ds(0, CH)], gbbuf.at[b], gg[b]).wait()

                for j in range(CH // 16):
                    dstb[b, pl.ds(16 * j, 16)] = lax.shift_right_logical(
                        idst[pl.ds(ci * CH + 16 * j, 16)], 3)

                @pl.when(ci >= 2)
                def _():
                    pltpu.make_async_copy(
                        msgbuf.at[b], accum.at[pl.ds(0, CH)], ss[b]).wait()

                @plsc.parallel_loop(0, CH, unroll=2)
                def _(e):
                    for j in range(DH // 32):
                        gv = plsc.bitcast(
                            gbbuf[b, e, pl.ds(DH // 2 + 16 * j, 16)],
                            jnp.bfloat16)
                        bv = plsc.bitcast(
                            gbbuf[b, e, pl.ds(16 * j, 16)], jnp.bfloat16)
                        xv = plsc.bitcast(
                            xbuf[b, e, pl.ds(16 * j, 16)], jnp.bfloat16)
                        g0, g1 = plsc.unpack(
                            gv, format=plsc.PackFormat.INTERLEAVED,
                            preferred_element_type=jnp.float32)
                        b0, b1 = plsc.unpack(
                            bv, format=plsc.PackFormat.INTERLEAVED,
                            preferred_element_type=jnp.float32)
                        x0, x1 = plsc.unpack(
                            xv, format=plsc.PackFormat.INTERLEAVED,
                            preferred_element_type=jnp.float32)
                        msgbuf[b, e, pl.ds(32 * j, 16)] = jnp.maximum(
                            g0 * x0 + b0, 0.0)
                        msgbuf[b, e, pl.ds(32 * j + 16, 16)] = jnp.maximum(
                            g1 * x1 + b1, 0.0)

                pltpu.async_copy(
                    msgbuf.at[b], accum.at[dstb.at[b]], ss[b], add=True)

                @pl.when(ci + 2 < NCHE)
                def _():
                    fire(ci + 2, b)

            return 0

        lax.fori_loop(0, NCHE // 2, body, 0)
        for b in range(2):
            pltpu.make_async_copy(
                msgbuf.at[b], accum.at[pl.ds(0, CH)], ss[b]).wait()
        plsc.subcore_barrier()
        pltpu.sync_copy(accum.at[pl.ds(rbase, RPT)],
                        out_hbm.at[pl.ds(rbase, RPT), pl.ds(c * DH, DH)])

    return k(xl_tab, gb_tab, srcr8, dstr8)


def _lohi(w64):
    lo = jnp.concatenate([w64[..., 0:16], w64[..., 32:48]], axis=-1)
    hi = jnp.concatenate([w64[..., 16:32], w64[..., 48:64]], axis=-1)
    return lo, hi


def kernel(x, edge_index, edge_type, mask, params):
    src = edge_index[0].astype(jnp.int32)
    dst = edge_index[1].astype(jnp.int32)
    rt = edge_type.astype(jnp.int32)
    srcr8 = src * (2 * R) + rt
    dstr8 = dst * (2 * R) + rt
    dstr4 = dst * R + rt

    cnt = _count_sc(dstr4)
    inv_exp = _prep_inv(cnt[0], cnt[1])
    iv = inv_exp.reshape(N, 4 * DH)
    inv_n = jnp.concatenate([iv, iv], axis=1)

    p = params
    h = _enc(x, p["enc"]["W"].T, p["enc"]["b"])

    gp = p["gru_s"]
    wa = jnp.concatenate(
        [gp["z0"]["W"].T, gp["r0"]["W"].T, gp["h0"]["W"].T], axis=1)
    wx = jnp.concatenate([gp["z1"]["W"].T, gp["r1"]["W"].T], axis=1)
    ws = jnp.concatenate(
        [gp["z2"]["W"].T, gp["r2"]["W"].T, gp["h2"]["W"].T], axis=1)
    wh1 = gp["h1"]["W"].T
    gbias = jnp.concatenate([
        gp["z0"]["b"] + gp["z1"]["b"] + gp["z2"]["b"],
        gp["r0"]["b"] + gp["r1"]["b"] + gp["r2"]["b"],
        gp["h0"]["b"] + gp["h1"]["b"] + gp["h2"]["b"],
    ]).reshape(1, 3 * D)

    for step in range(NSTEP):
        cp = p["conv"][step]
        xl_lo, xl_hi = [], []
        for c in range(NC):
            for r in range(R):
                lo, hi = _lohi(cp["lins"][r]["W"].T[:, c * DH:(c + 1) * DH])
                xl_lo.append(lo)
                xl_hi.append(hi)
        gb_lo, gb_hi, bias_lo, bias_hi = [], [], [], []
        for c in range(NC):
            for r in range(R):
                wt = cp["films"][r]["W"].T
                bb = cp["films"][r]["b"]
                for piece in (wt[:, c * DH:(c + 1) * DH],
                              wt[:, D + c * DH:D + (c + 1) * DH]):
                    lo, hi = _lohi(piece)
                    gb_lo.append(lo)
                    gb_hi.append(hi)
                for piece in (bb[c * DH:(c + 1) * DH],
                              bb[D + c * DH:D + (c + 1) * DH]):
                    lo, hi = _lohi(piece)
                    bias_lo.append(lo)
                    bias_hi.append(hi)
        w_cat = jnp.concatenate(
            [cp["lin_skip"]["W"].T, cp["film_skip"]["W"].T]
            + xl_lo + xl_hi + gb_lo + gb_hi,
            axis=1)
        bias_cat = jnp.concatenate(
            bias_lo + bias_hi).reshape(1, 8 * D)

        skip, xl, gb = _pre(h, w_cat, bias_cat, inv_n)
        xl_tab = xl.reshape(2 * NR, DH // 2)
        gb_tab = gb.reshape(2 * NR, DH)

        q = _attn(h.reshape(B, L, D), mask)
        sn = jnp.repeat(q, L, axis=0)

        part = _edges_sc(xl_tab, gb_tab, srcr8, dstr8)
        h = _gru(h, skip, part, sn, wa, wx, ws, wh1, gbias)

    return h.reshape(B, L, D)


# speedup vs baseline: 32.4683x; 1.0009x over previous
import functools
import math

import jax
import jax.numpy as jnp
from jax import lax
from jax.experimental import pallas as pl
from jax.experimental.pallas import tpu as pltpu
from jax.experimental.pallas import tpu_sc as plsc

N = 10000
B = 100
L = 100
D = 128
E = 320000
R = 4
NSTEP = 2
NR = N * R

NC = 2
NS = 16
DH = D // NC
EPT = E // NS
CH = 80
NCHE = EPT // CH
NW = NC * NS
EPW = E // NW
NCHC = EPW // CH
RPT = N // NS
CRP = NR // NS


def _enc_body(x_ref, w_ref, b_ref, o_ref):
    y = jnp.dot(x_ref[...], w_ref[...], preferred_element_type=jnp.float32)
    o_ref[...] = jnp.maximum(y + b_ref[...], 0.0)


def _enc(x, w_t, b):
    blk = 2000
    return pl.pallas_call(
        _enc_body,
        grid=(N // blk,),
        in_specs=[
            pl.BlockSpec((blk, D), lambda i: (i, 0)),
            pl.BlockSpec((D, D), lambda i: (0, 0)),
            pl.BlockSpec((1, D), lambda i: (0, 0)),
        ],
        out_specs=pl.BlockSpec((blk, D), lambda i: (i, 0)),
        out_shape=jax.ShapeDtypeStruct((N, D), jnp.float32),
    )(x, w_t, b.reshape(1, D))


def _inv_body(c0_ref, c1_ref, o_ref):
    s = c0_ref[:, 0:1] + c1_ref[:, 0:1]
    inv = 1.0 / jnp.maximum(s, 1.0)
    o_ref[...] = jnp.broadcast_to(inv, o_ref.shape)


def _prep_inv(cnt0, cnt1):
    blk = 4000
    return pl.pallas_call(
        _inv_body,
        grid=(NR // blk,),
        in_specs=[
            pl.BlockSpec((blk, 16), lambda i: (i, 0)),
            pl.BlockSpec((blk, 16), lambda i: (i, 0)),
        ],
        out_specs=pl.BlockSpec((blk, DH), lambda i: (i, 0)),
        out_shape=jax.ShapeDtypeStruct((NR, DH), jnp.float32),
    )(cnt0, cnt1)


def _pack_pair(lo, hi):
    lo_b = lax.bitcast_convert_type(lo.astype(jnp.bfloat16), jnp.uint16)
    hi_b = lax.bitcast_convert_type(hi.astype(jnp.bfloat16), jnp.uint16)
    u = lo_b.astype(jnp.uint32) | (hi_b.astype(jnp.uint32) << 16)
    return lax.bitcast_convert_type(u, jnp.float32)


def _pre_body(h_ref, w_ref, bias_ref, inv_ref, skip_ref, xl_ref, gb_ref):
    y = jnp.dot(h_ref[...], w_ref[...], preferred_element_type=jnp.float32)
    skip_ref[...] = jnp.maximum(y[:, 256:384] * y[:, 0:128] + y[:, 128:256], 0.0)
    xlp = _pack_pair(y[:, 384:640], y[:, 640:896])
    xl_ref[:, 0, :] = xlp[:, 0:128]
    xl_ref[:, 1, :] = xlp[:, 128:256]
    inv = inv_ref[...]
    bias = bias_ref[...]
    glo = (y[:, 896:1408] + bias[:, 0:512]) * inv
    ghi = (y[:, 1408:1920] + bias[:, 512:1024]) * inv
    gbp = _pack_pair(glo, ghi)
    gb_ref[:, 0, :] = gbp[:, 0:128]
    gb_ref[:, 1, :] = gbp[:, 128:256]
    gb_ref[:, 2, :] = gbp[:, 256:384]
    gb_ref[:, 3, :] = gbp[:, 384:512]


def _pre(h, w_cat, bias_cat, inv_n):
    blk = 2000
    return pl.pallas_call(
        _pre_body,
        grid=(N // blk,),
        in_specs=[
            pl.BlockSpec((blk, D), lambda i: (i, 0)),
            pl.BlockSpec((D, 15 * D), lambda i: (0, 0)),
            pl.BlockSpec((1, 8 * D), lambda i: (0, 0)),
            pl.BlockSpec((blk, 4 * D), lambda i: (i, 0)),
        ],
        out_specs=[
            pl.BlockSpec((blk, D), lambda i: (i, 0)),
            pl.BlockSpec((blk, 2, D), lambda i: (i, 0, 0)),
            pl.BlockSpec((blk, 4, D), lambda i: (i, 0, 0)),
        ],
        out_shape=[
            jax.ShapeDtypeStruct((N, D), jnp.float32),
            jax.ShapeDtypeStruct((N, 2, D), jnp.float32),
            jax.ShapeDtypeStruct((N, 4, D), jnp.float32),
        ],
    )(h, w_cat, bias_cat, inv_n)


_AG = 10


def _attn_body(h_ref, m_ref, q_ref):
    for g in range(_AG):
        hm = h_ref[g] * m_ref[g]
        logits = lax.dot_general(
            hm, hm, (((1,), (1,)), ((), ())),
            preferred_element_type=jnp.float32) * (1.0 / math.sqrt(D))
        mx = jnp.max(logits, axis=-1, keepdims=True)
        e = jnp.exp(logits - mx)
        aw = e / jnp.sum(e, axis=-1, keepdims=True)
        a = jnp.dot(aw, hm, preferred_element_type=jnp.float32)
        q_ref[g, 0] = jnp.mean(a, axis=0)


def _attn(h3, mask):
    out = pl.pallas_call(
        _attn_body,
        grid=(B // _AG,),
        in_specs=[
            pl.BlockSpec((_AG, L, D), lambda i: (i, 0, 0)),
            pl.BlockSpec((_AG, L, 1), lambda i: (i, 0, 0)),
        ],
        out_specs=pl.BlockSpec((_AG, 1, D), lambda i: (i, 0, 0)),
        out_shape=jax.ShapeDtypeStruct((B, 1, D), jnp.float32),
    )(h3, mask)
    return out.reshape(B, D)


def _gru_body(h_ref, skip_ref, p_ref, sn_ref, wa_ref, wx_ref,
              ws_ref, wh1_ref, bias_ref, o_ref):
    h = h_ref[...]
    a = skip_ref[...] + p_ref[...]
    sn = sn_ref[...]
    ya = jnp.dot(a, wa_ref[...], preferred_element_type=jnp.float32)
    yx = jnp.dot(h, wx_ref[...], preferred_element_type=jnp.float32)
    ys = jnp.dot(sn, ws_ref[...], preferred_element_type=jnp.float32)
    bias = bias_ref[...]
    z = jax.nn.sigmoid(ya[:, 0:D] + yx[:, 0:D] + ys[:, 0:D] + bias[:, 0:D])
    r = jax.nn.sigmoid(
        ya[:, D:2 * D] + yx[:, D:2 * D] + ys[:, D:2 * D] + bias[:, D:2 * D])
    hr = jnp.dot(h * r, wh1_ref[...], preferred_element_type=jnp.float32)
    hh = jnp.maximum(
        ya[:, 2 * D:3 * D] + hr + ys[:, 2 * D:3 * D] + bias[:, 2 * D:3 * D], 0.0)
    o_ref[...] = hh * z + h * (1.0 - z)


def _gru(h, skip, p, sn, wa, wx, ws, wh1, bias):
    blk = 2000
    return pl.pallas_call(
        _gru_body,
        grid=(N // blk,),
        in_specs=[
            pl.BlockSpec((blk, D), lambda i: (i, 0)),
            pl.BlockSpec((blk, D), lambda i: (i, 0)),
            pl.BlockSpec((blk, D), lambda i: (i, 0)),
            pl.BlockSpec((blk, D), lambda i: (i, 0)),
            pl.BlockSpec((D, 3 * D), lambda i: (0, 0)),
            pl.BlockSpec((D, 2 * D), lambda i: (0, 0)),
            pl.BlockSpec((D, 3 * D), lambda i: (0, 0)),
            pl.BlockSpec((D, D), lambda i: (0, 0)),
            pl.BlockSpec((1, 3 * D), lambda i: (0, 0)),
        ],
        out_specs=pl.BlockSpec((blk, D), lambda i: (i, 0)),
        out_shape=jax.ShapeDtypeStruct((N, D), jnp.float32),
    )(h, skip, p, sn, wa, wx, ws, wh1, bias)


def _count_sc(dstr4):
    mesh = plsc.VectorSubcoreMesh(core_axis_name="c", subcore_axis_name="s")

    @functools.partial(
        pl.kernel,
        out_type=jax.ShapeDtypeStruct((NC, NR, 16), jnp.float32),
        mesh=mesh,
        compiler_params=pltpu.CompilerParams(use_tc_tiling_on_sc=False),
        scratch_types=[
            pltpu.VMEM((EPW,), jnp.int32),
            pltpu.VMEM((NCHC, CH), jnp.int32),
            pltpu.VMEM((CH, 16), jnp.float32),
            pltpu.VMEM((100, 16), jnp.float32),
            pltpu.VMEM_SHARED((NR, 16), jnp.float32),
            pltpu.SemaphoreType.DMA,
        ],
    )
    def k(dstr_hbm, out_hbm, dbuf, ibuf2d, ones_v, zbuf, cnt_sh, sem):
        c = lax.axis_index("c")
        s = lax.axis_index("s")
        wid = s * NC + c
        lanes = lax.iota(jnp.int32, 16)
        unit = jnp.where(lanes == 0, 1.0, 0.0).astype(jnp.float32)
        zero = jnp.zeros((16,), jnp.float32)

        pltpu.sync_copy(dstr_hbm.at[pl.ds(wid * EPW, EPW)], dbuf)

        def mkidx(ci, _):
            for j in range(CH // 16):
                ibuf2d[ci, pl.ds(16 * j, 16)] = dbuf[pl.ds(ci * CH + 16 * j, 16)]
            return 0

        lax.fori_loop(0, NCHC, mkidx, 0)

        def fill(i, _):
            ones_v[i, :] = unit
            return 0

        lax.fori_loop(0, CH, fill, 0)

        def fillz(i, _):
            zbuf[i, :] = zero
            return 0

        lax.fori_loop(0, 100, fillz, 0)

        def zcp(i, _):
            pltpu.sync_copy(zbuf, cnt_sh.at[pl.ds(s * CRP + i * 100, 100)])
            return 0

        lax.fori_loop(0, CRP // 100, zcp, 0)
        plsc.subcore_barrier()

        def chunk(i, _):
            pltpu.async_copy(ones_v, cnt_sh.at[ibuf2d.at[i]], sem, add=True)

            @pl.when(i >= 4)
            def _():
                pltpu.make_async_copy(
                    ones_v, cnt_sh.at[pl.ds(0, CH)], sem).wait()

            return 0

        lax.fori_loop(0, NCHC, chunk, 0)
        for _ in range(4):
            pltpu.make_async_copy(ones_v, cnt_sh.at[pl.ds(0, CH)], sem).wait()
        plsc.subcore_barrier()

        def dump(i, _):
            sl = pl.ds(s * CRP + i * 100, 100)
            pltpu.sync_copy(cnt_sh.at[sl], out_hbm.at[c, sl])
            return 0

        lax.fori_loop(0, CRP // 100, dump, 0)

    return k(dstr4)


def _edges_sc(xl_tab, gb_tab, srcr8, dstr8):
    mesh = plsc.VectorSubcoreMesh(core_axis_name="c", subcore_axis_name="s")

    @functools.partial(
        pl.kernel,
        out_type=jax.ShapeDtypeStruct((N, D), jnp.float32),
        mesh=mesh,
        compiler_params=pltpu.CompilerParams(
            use_tc_tiling_on_sc=False, needs_layout_passes=False),
        scratch_types=[
            pltpu.VMEM((EPT,), jnp.int32),
            pltpu.VMEM((EPT,), jnp.int32),
            pltpu.VMEM((2, CH), jnp.int32),
            pltpu.VMEM((2, CH, DH // 2), jnp.float32),
            pltpu.VMEM((2, CH, DH), jnp.float32),
            pltpu.VMEM((2, CH, DH), jnp.float32),
            pltpu.VMEM_SHARED((N, DH), jnp.float32),
            pltpu.SemaphoreType.DMA,
            pltpu.SemaphoreType.DMA,
            pltpu.SemaphoreType.DMA,
            pltpu.SemaphoreType.DMA,
            pltpu.SemaphoreType.DMA,
            pltpu.SemaphoreType.DMA,
        ],
    )
    def k(xl_hbm, gb_hbm, srcr_hbm, dstr_hbm, out_hbm,
          isrc, idst, dstb, xbuf, gbbuf, msgbuf, accum,
          gx0, gx1, gg0, gg1, ss0, ss1):
        gx = (gx0, gx1)
        gg = (gg0, gg1)
        ss = (ss0, ss1)
        c = lax.axis_index("c")
        s = lax.axis_index("s")
        ebase = s * EPT
        zero = jnp.zeros((16,), jnp.float32)

        pltpu.sync_copy(srcr_hbm.at[pl.ds(ebase, EPT)], isrc)
        pltpu.sync_copy(dstr_hbm.at[pl.ds(ebase, EPT)], idst)
        off = c * R

        def adj(i, _):
            sl = pl.ds(16 * i, 16)
            isrc[sl] = isrc[sl] + off
            idst[sl] = idst[sl] + off
            return 0

        lax.fori_loop(0, EPT // 16, adj, 0)

        def zb(i, _):
            for j in range(DH // 16):
                msgbuf[0, i, pl.ds(j * 16, 16)] = zero
            return 0

        lax.fori_loop(0, CH, zb, 0)
        rbase = s * RPT
        for i in range(RPT // CH):
            pltpu.sync_copy(msgbuf.at[0], accum.at[pl.ds(rbase + i * CH, CH)])
        rem = RPT - (RPT // CH) * CH
        pltpu.sync_copy(msgbuf.at[0, pl.ds(0, rem)],
                        accum.at[pl.ds(rbase + RPT - rem, rem)])
        plsc.subcore_barrier()

        def fire(ci, b):
            pltpu.async_copy(
                xl_hbm.at[isrc.at[pl.ds(ci * CH, CH)]], xbuf.at[b], gx[b])
            pltpu.async_copy(
                gb_hbm.at[idst.at[pl.ds(ci * CH, CH)]], gbbuf.at[b], gg[b])

        fire(0, 0)
        fire(1, 1)

        def body(i, _):
            for b in range(2):
                ci = 2 * i + b
                pltpu.make_async_copy(
                    xl_hbm.at[pl.ds(0, CH)], xbuf.at[b], gx[b]).wait()
                pltpu.make_async_copy(
                    gb_hbm.at[pl.ds(0, CH)], gbbuf.at[b], gg[b]).wait()

                for j in range(CH // 16):
                    dstb[b, pl.ds(16 * j, 16)] = lax.shift_right_logical(
                        idst[pl.ds(ci * CH + 16 * j, 16)], 3)

                @pl.when(ci >= 2)
                def _():
                    pltpu.make_async_copy(
                        msgbuf.at[b], accum.at[pl.ds(0, CH)], ss[b]).wait()

                @plsc.parallel_loop(0, CH, unroll=2)
                def _(e):
                    for j in range(DH // 32):
                        gv = plsc.bitcast(
                            gbbuf[b, e, pl.ds(DH // 2 + 16 * j, 16)],
                            jnp.bfloat16)
                        bv = plsc.bitcast(
                            gbbuf[b, e, pl.ds(16 * j, 16)], jnp.bfloat16)
                        xv = plsc.bitcast(
                            xbuf[b, e, pl.ds(16 * j, 16)], jnp.bfloat16)
                        g0, g1 = plsc.unpack(
                            gv, format=plsc.PackFormat.INTERLEAVED,
                            preferred_element_type=jnp.float32)
                        b0, b1 = plsc.unpack(
                            bv, format=plsc.PackFormat.INTERLEAVED,
                            preferred_element_type=jnp.float32)
                        x0, x1 = plsc.unpack(
                            xv, format=plsc.PackFormat.INTERLEAVED,
                            preferred_element_type=jnp.float32)
                        msgbuf[b, e, pl.ds(32 * j, 16)] = jnp.maximum(
                            g0 * x0 + b0, 0.0)
                        msgbuf[b, e, pl.ds(32 * j + 16, 16)] = jnp.maximum(
                            g1 * x1 + b1, 0.0)

                pltpu.async_copy(
                    msgbuf.at[b], accum.at[dstb.at[b]], ss[b], add=True)

                @pl.when(ci + 2 < NCHE)
                def _():
                    fire(ci + 2, b)

            return 0

        lax.fori_loop(0, NCHE // 2, body, 0)
        for b in range(2):
            pltpu.make_async_copy(
                msgbuf.at[b], accum.at[pl.ds(0, CH)], ss[b]).wait()
        plsc.subcore_barrier()
        pltpu.sync_copy(accum.at[pl.ds(rbase, RPT)],
                        out_hbm.at[pl.ds(rbase, RPT), pl.ds(c * DH, DH)])

    return k(xl_tab, gb_tab, srcr8, dstr8)


def _lohi(w64):
    lo = jnp.concatenate([w64[..., 0:16], w64[..., 32:48]], axis=-1)
    hi = jnp.concatenate([w64[..., 16:32], w64[..., 48:64]], axis=-1)
    return lo, hi


def kernel(x, edge_index, edge_type, mask, params):
    src = edge_index[0].astype(jnp.int32)
    dst = edge_index[1].astype(jnp.int32)
    rt = edge_type.astype(jnp.int32)
    srcr8 = src * (2 * R) + rt
    dstr8 = dst * (2 * R) + rt
    dstr4 = dst * R + rt

    cnt = _count_sc(dstr4)
    inv_exp = _prep_inv(cnt[0], cnt[1])
    iv = inv_exp.reshape(N, 4 * DH)
    inv_n = jnp.concatenate([iv, iv], axis=1)

    p = params
    h = _enc(x, p["enc"]["W"].T, p["enc"]["b"])

    gp = p["gru_s"]
    wa = jnp.concatenate(
        [gp["z0"]["W"].T, gp["r0"]["W"].T, gp["h0"]["W"].T], axis=1)
    wx = jnp.concatenate([gp["z1"]["W"].T, gp["r1"]["W"].T], axis=1)
    ws = jnp.concatenate(
        [gp["z2"]["W"].T, gp["r2"]["W"].T, gp["h2"]["W"].T], axis=1)
    wh1 = gp["h1"]["W"].T
    gbias = jnp.concatenate([
        gp["z0"]["b"] + gp["z1"]["b"] + gp["z2"]["b"],
        gp["r0"]["b"] + gp["r1"]["b"] + gp["r2"]["b"],
        gp["h0"]["b"] + gp["h1"]["b"] + gp["h2"]["b"],
    ]).reshape(1, 3 * D)

    for step in range(NSTEP):
        cp = p["conv"][step]
        xl_lo, xl_hi = [], []
        for c in range(NC):
            for r in range(R):
                lo, hi = _lohi(cp["lins"][r]["W"].T[:, c * DH:(c + 1) * DH])
                xl_lo.append(lo)
                xl_hi.append(hi)
        gb_lo, gb_hi, bias_lo, bias_hi = [], [], [], []
        for c in range(NC):
            for r in range(R):
                wt = cp["films"][r]["W"].T
                bb = cp["films"][r]["b"]
                for piece in (wt[:, c * DH:(c + 1) * DH],
                              wt[:, D + c * DH:D + (c + 1) * DH]):
                    lo, hi = _lohi(piece)
                    gb_lo.append(lo)
                    gb_hi.append(hi)
                for piece in (bb[c * DH:(c + 1) * DH],
                              bb[D + c * DH:D + (c + 1) * DH]):
                    lo, hi = _lohi(piece)
                    bias_lo.append(lo)
                    bias_hi.append(hi)
        w_cat = jnp.concatenate(
            [cp["lin_skip"]["W"].T, cp["film_skip"]["W"].T]
            + xl_lo + xl_hi + gb_lo + gb_hi,
            axis=1)
        bias_cat = jnp.concatenate(
            bias_lo + bias_hi).reshape(1, 8 * D)

        skip, xl, gb = _pre(h, w_cat, bias_cat, inv_n)
        xl_tab = xl.reshape(2 * NR, DH // 2)
        gb_tab = gb.reshape(2 * NR, DH)

        part = _edges_sc(xl_tab, gb_tab, srcr8, dstr8)
        q = _attn(h.reshape(B, L, D), mask)
        sn = jnp.repeat(q, L, axis=0)
        h = _gru(h, skip, part, sn, wa, wx, ws, wh1, gbias)

    return h.reshape(B, L, D)


# speedup vs baseline: 37.0481x; 1.1411x over previous
import functools
import math

import jax
import jax.numpy as jnp
from jax import lax
from jax.experimental import pallas as pl
from jax.experimental.pallas import tpu as pltpu
from jax.experimental.pallas import tpu_sc as plsc

N = 10000
B = 100
L = 100
D = 128
E = 320000
R = 4
NSTEP = 2
NR = N * R

NC = 2
NS = 16
DH = D // NC
EPT = E // NS
CH = 80
NCHE = EPT // CH
NW = NC * NS
EPW = E // NW
NCHC = EPW // CH
RPT = N // NS
CRP = NR // NS


def _enc_body(x_ref, w_ref, b_ref, o_ref):
    y = jnp.dot(x_ref[...], w_ref[...], preferred_element_type=jnp.float32)
    o_ref[...] = jnp.maximum(y + b_ref[...], 0.0)


def _enc(x, w_t, b):
    blk = 2000
    return pl.pallas_call(
        _enc_body,
        grid=(N // blk,),
        in_specs=[
            pl.BlockSpec((blk, D), lambda i: (i, 0)),
            pl.BlockSpec((D, D), lambda i: (0, 0)),
            pl.BlockSpec((1, D), lambda i: (0, 0)),
        ],
        out_specs=pl.BlockSpec((blk, D), lambda i: (i, 0)),
        out_shape=jax.ShapeDtypeStruct((N, D), jnp.float32),
    )(x, w_t, b.reshape(1, D))


def _inv_body(c0_ref, c1_ref, o_ref):
    s = c0_ref[:, 0:1] + c1_ref[:, 0:1]
    inv = 1.0 / jnp.maximum(s, 1.0)
    o_ref[...] = jnp.broadcast_to(inv, o_ref.shape)


def _prep_inv(cnt0, cnt1):
    blk = 4000
    return pl.pallas_call(
        _inv_body,
        grid=(NR // blk,),
        in_specs=[
            pl.BlockSpec((blk, 16), lambda i: (i, 0)),
            pl.BlockSpec((blk, 16), lambda i: (i, 0)),
        ],
        out_specs=pl.BlockSpec((blk, DH), lambda i: (i, 0)),
        out_shape=jax.ShapeDtypeStruct((NR, DH), jnp.float32),
    )(cnt0, cnt1)


def _pack_pair(lo, hi):
    lo_b = lax.bitcast_convert_type(lo.astype(jnp.bfloat16), jnp.uint16)
    hi_b = lax.bitcast_convert_type(hi.astype(jnp.bfloat16), jnp.uint16)
    u = lo_b.astype(jnp.uint32) | (hi_b.astype(jnp.uint32) << 16)
    return lax.bitcast_convert_type(u, jnp.float32)


def _pre_body(h_ref, w_ref, bias_ref, inv_ref, skip_ref, xl_ref, gb_ref):
    y = jnp.dot(h_ref[...], w_ref[...], preferred_element_type=jnp.float32)
    skip_ref[...] = jnp.maximum(y[:, 256:384] * y[:, 0:128] + y[:, 128:256], 0.0)
    xlp = _pack_pair(y[:, 384:640], y[:, 640:896])
    xl_ref[:, 0, :] = xlp[:, 0:128]
    xl_ref[:, 1, :] = xlp[:, 128:256]
    inv = inv_ref[...]
    bias = bias_ref[...]
    glo = (y[:, 896:1408] + bias[:, 0:512]) * inv
    ghi = (y[:, 1408:1920] + bias[:, 512:1024]) * inv
    gbp = _pack_pair(glo, ghi)
    gb_ref[:, 0, :] = gbp[:, 0:128]
    gb_ref[:, 1, :] = gbp[:, 128:256]
    gb_ref[:, 2, :] = gbp[:, 256:384]
    gb_ref[:, 3, :] = gbp[:, 384:512]


def _pre(h, w_cat, bias_cat, inv_n):
    blk = 2000
    return pl.pallas_call(
        _pre_body,
        grid=(N // blk,),
        in_specs=[
            pl.BlockSpec((blk, D), lambda i: (i, 0)),
            pl.BlockSpec((D, 15 * D), lambda i: (0, 0)),
            pl.BlockSpec((1, 8 * D), lambda i: (0, 0)),
            pl.BlockSpec((blk, 4 * D), lambda i: (i, 0)),
        ],
        out_specs=[
            pl.BlockSpec((blk, D), lambda i: (i, 0)),
            pl.BlockSpec((blk, 2, D), lambda i: (i, 0, 0)),
            pl.BlockSpec((blk, 4, D), lambda i: (i, 0, 0)),
        ],
        out_shape=[
            jax.ShapeDtypeStruct((N, D), jnp.float32),
            jax.ShapeDtypeStruct((N, 2, D), jnp.float32),
            jax.ShapeDtypeStruct((N, 4, D), jnp.float32),
        ],
    )(h, w_cat, bias_cat, inv_n)


_AG = 10


def _attn_body(h_ref, m_ref, q_ref):
    for g in range(_AG):
        hm = h_ref[g] * m_ref[g]
        logits = lax.dot_general(
            hm, hm, (((1,), (1,)), ((), ())),
            preferred_element_type=jnp.float32) * (1.0 / math.sqrt(D))
        mx = jnp.max(logits, axis=-1, keepdims=True)
        e = jnp.exp(logits - mx)
        aw = e / jnp.sum(e, axis=-1, keepdims=True)
        a = jnp.dot(aw, hm, preferred_element_type=jnp.float32)
        q_ref[g, 0] = jnp.mean(a, axis=0)


def _attn(h3, mask):
    out = pl.pallas_call(
        _attn_body,
        grid=(B // _AG,),
        in_specs=[
            pl.BlockSpec((_AG, L, D), lambda i: (i, 0, 0)),
            pl.BlockSpec((_AG, L, 1), lambda i: (i, 0, 0)),
        ],
        out_specs=pl.BlockSpec((_AG, 1, D), lambda i: (i, 0, 0)),
        out_shape=jax.ShapeDtypeStruct((B, 1, D), jnp.float32),
    )(h3, mask)
    return out.reshape(B, D)


def _gru_body(h_ref, skip_ref, p_ref, sn_ref, wa_ref, wx_ref,
              ws_ref, wh1_ref, bias_ref, o_ref):
    h = h_ref[...]
    a = skip_ref[...] + p_ref[...]
    sn = sn_ref[...]
    ya = jnp.dot(a, wa_ref[...], preferred_element_type=jnp.float32)
    yx = jnp.dot(h, wx_ref[...], preferred_element_type=jnp.float32)
    ys = jnp.dot(sn, ws_ref[...], preferred_element_type=jnp.float32)
    bias = bias_ref[...]
    z = jax.nn.sigmoid(ya[:, 0:D] + yx[:, 0:D] + ys[:, 0:D] + bias[:, 0:D])
    r = jax.nn.sigmoid(
        ya[:, D:2 * D] + yx[:, D:2 * D] + ys[:, D:2 * D] + bias[:, D:2 * D])
    hr = jnp.dot(h * r, wh1_ref[...], preferred_element_type=jnp.float32)
    hh = jnp.maximum(
        ya[:, 2 * D:3 * D] + hr + ys[:, 2 * D:3 * D] + bias[:, 2 * D:3 * D], 0.0)
    o_ref[...] = hh * z + h * (1.0 - z)


def _gru(h, skip, p, sn, wa, wx, ws, wh1, bias):
    blk = 2000
    return pl.pallas_call(
        _gru_body,
        grid=(N // blk,),
        in_specs=[
            pl.BlockSpec((blk, D), lambda i: (i, 0)),
            pl.BlockSpec((blk, D), lambda i: (i, 0)),
            pl.BlockSpec((blk, D), lambda i: (i, 0)),
            pl.BlockSpec((blk, D), lambda i: (i, 0)),
            pl.BlockSpec((D, 3 * D), lambda i: (0, 0)),
            pl.BlockSpec((D, 2 * D), lambda i: (0, 0)),
            pl.BlockSpec((D, 3 * D), lambda i: (0, 0)),
            pl.BlockSpec((D, D), lambda i: (0, 0)),
            pl.BlockSpec((1, 3 * D), lambda i: (0, 0)),
        ],
        out_specs=pl.BlockSpec((blk, D), lambda i: (i, 0)),
        out_shape=jax.ShapeDtypeStruct((N, D), jnp.float32),
    )(h, skip, p, sn, wa, wx, ws, wh1, bias)


def _count_sc(dstr4):
    mesh = plsc.VectorSubcoreMesh(core_axis_name="c", subcore_axis_name="s")

    @functools.partial(
        pl.kernel,
        out_type=jax.ShapeDtypeStruct((NC, NR, 16), jnp.float32),
        mesh=mesh,
        compiler_params=pltpu.CompilerParams(use_tc_tiling_on_sc=False),
        scratch_types=[
            pltpu.VMEM((EPW,), jnp.int32),
            pltpu.VMEM((NCHC, CH), jnp.int32),
            pltpu.VMEM((CH, 16), jnp.float32),
            pltpu.VMEM((100, 16), jnp.float32),
            pltpu.VMEM_SHARED((NR, 16), jnp.float32),
            pltpu.SemaphoreType.DMA,
        ],
    )
    def k(dstr_hbm, out_hbm, dbuf, ibuf2d, ones_v, zbuf, cnt_sh, sem):
        c = lax.axis_index("c")
        s = lax.axis_index("s")
        wid = s * NC + c
        lanes = lax.iota(jnp.int32, 16)
        unit = jnp.where(lanes == 0, 1.0, 0.0).astype(jnp.float32)
        zero = jnp.zeros((16,), jnp.float32)

        pltpu.sync_copy(dstr_hbm.at[pl.ds(wid * EPW, EPW)], dbuf)

        def mkidx(ci, _):
            for j in range(CH // 16):
                ibuf2d[ci, pl.ds(16 * j, 16)] = dbuf[pl.ds(ci * CH + 16 * j, 16)]
            return 0

        lax.fori_loop(0, NCHC, mkidx, 0)

        def fill(i, _):
            ones_v[i, :] = unit
            return 0

        lax.fori_loop(0, CH, fill, 0)

        def fillz(i, _):
            zbuf[i, :] = zero
            return 0

        lax.fori_loop(0, 100, fillz, 0)

        def zcp(i, _):
            pltpu.sync_copy(zbuf, cnt_sh.at[pl.ds(s * CRP + i * 100, 100)])
            return 0

        lax.fori_loop(0, CRP // 100, zcp, 0)
        plsc.subcore_barrier()

        def chunk(i, _):
            pltpu.async_copy(ones_v, cnt_sh.at[ibuf2d.at[i]], sem, add=True)

            @pl.when(i >= 4)
            def _():
                pltpu.make_async_copy(
                    ones_v, cnt_sh.at[pl.ds(0, CH)], sem).wait()

            return 0

        lax.fori_loop(0, NCHC, chunk, 0)
        for _ in range(4):
            pltpu.make_async_copy(ones_v, cnt_sh.at[pl.ds(0, CH)], sem).wait()
        plsc.subcore_barrier()

        def dump(i, _):
            sl = pl.ds(s * CRP + i * 100, 100)
            pltpu.sync_copy(cnt_sh.at[sl], out_hbm.at[c, sl])
            return 0

        lax.fori_loop(0, CRP // 100, dump, 0)

    return k(dstr4)


def _edges_sc(xl_tab, gb_tab, srcr8, dstr8):
    mesh = plsc.VectorSubcoreMesh(core_axis_name="c", subcore_axis_name="s")

    @functools.partial(
        pl.kernel,
        out_type=jax.ShapeDtypeStruct((N, D), jnp.float32),
        mesh=mesh,
        compiler_params=pltpu.CompilerParams(
            use_tc_tiling_on_sc=False, needs_layout_passes=False),
        scratch_types=[
            pltpu.VMEM((EPT,), jnp.int32),
            pltpu.VMEM((EPT,), jnp.int32),
            pltpu.VMEM((3, CH), jnp.int32),
            pltpu.VMEM((3, CH, DH // 2), jnp.float32),
            pltpu.VMEM((3, CH, DH), jnp.float32),
            pltpu.VMEM((3, CH, DH), jnp.float32),
            pltpu.VMEM_SHARED((N, DH), jnp.float32),
            pltpu.SemaphoreType.DMA,
            pltpu.SemaphoreType.DMA,
            pltpu.SemaphoreType.DMA,
            pltpu.SemaphoreType.DMA,
            pltpu.SemaphoreType.DMA,
            pltpu.SemaphoreType.DMA,
            pltpu.SemaphoreType.DMA,
            pltpu.SemaphoreType.DMA,
            pltpu.SemaphoreType.DMA,
        ],
    )
    def k(xl_hbm, gb_hbm, srcr_hbm, dstr_hbm, out_hbm,
          isrc, idst, dstb, xbuf, gbbuf, msgbuf, accum,
          gx0, gx1, gx2, gg0, gg1, gg2, ss0, ss1, ss2):
        gx = (gx0, gx1, gx2)
        gg = (gg0, gg1, gg2)
        ss = (ss0, ss1, ss2)
        c = lax.axis_index("c")
        s = lax.axis_index("s")
        ebase = s * EPT
        zero = jnp.zeros((16,), jnp.float32)

        pltpu.sync_copy(srcr_hbm.at[pl.ds(ebase, EPT)], isrc)
        pltpu.sync_copy(dstr_hbm.at[pl.ds(ebase, EPT)], idst)
        off = c * R

        def adj(i, _):
            sl = pl.ds(16 * i, 16)
            isrc[sl] = isrc[sl] + off
            idst[sl] = idst[sl] + off
            return 0

        lax.fori_loop(0, EPT // 16, adj, 0)

        def zb(i, _):
            for j in range(DH // 16):
                msgbuf[0, i, pl.ds(j * 16, 16)] = zero
            return 0

        lax.fori_loop(0, CH, zb, 0)
        rbase = s * RPT
        for i in range(RPT // CH):
            pltpu.sync_copy(msgbuf.at[0], accum.at[pl.ds(rbase + i * CH, CH)])
        rem = RPT - (RPT // CH) * CH
        pltpu.sync_copy(msgbuf.at[0, pl.ds(0, rem)],
                        accum.at[pl.ds(rbase + RPT - rem, rem)])
        plsc.subcore_barrier()

        def fire(ci, b):
            pltpu.async_copy(
                xl_hbm.at[isrc.at[pl.ds(ci * CH, CH)]], xbuf.at[b], gx[b])
            pltpu.async_copy(
                gb_hbm.at[idst.at[pl.ds(ci * CH, CH)]], gbbuf.at[b], gg[b])

        fire(0, 0)
        fire(1, 1)
        fire(2, 2)

        def body(i, _):
            for b in range(3):
                ci = 3 * i + b
                pltpu.make_async_copy(
                    xl_hbm.at[pl.ds(0, CH)], xbuf.at[b], gx[b]).wait()
                pltpu.make_async_copy(
                    gb_hbm.at[pl.ds(0, CH)], gbbuf.at[b], gg[b]).wait()

                for j in range(CH // 16):
                    dstb[b, pl.ds(16 * j, 16)] = lax.shift_right_logical(
                        idst[pl.ds(ci * CH + 16 * j, 16)], 3)

                @pl.when(ci >= 3)
                def _():
                    pltpu.make_async_copy(
                        msgbuf.at[b], accum.at[pl.ds(0, CH)], ss[b]).wait()

                @plsc.parallel_loop(0, CH, unroll=2)
                def _(e):
                    for j in range(DH // 32):
                        gv = plsc.bitcast(
                            gbbuf[b, e, pl.ds(DH // 2 + 16 * j, 16)],
                            jnp.bfloat16)
                        bv = plsc.bitcast(
                            gbbuf[b, e, pl.ds(16 * j, 16)], jnp.bfloat16)
                        xv = plsc.bitcast(
                            xbuf[b, e, pl.ds(16 * j, 16)], jnp.bfloat16)
                        g0, g1 = plsc.unpack(
                            gv, format=plsc.PackFormat.INTERLEAVED,
                            preferred_element_type=jnp.float32)
                        b0, b1 = plsc.unpack(
                            bv, format=plsc.PackFormat.INTERLEAVED,
                            preferred_element_type=jnp.float32)
                        x0, x1 = plsc.unpack(
                            xv, format=plsc.PackFormat.INTERLEAVED,
                            preferred_element_type=jnp.float32)
                        msgbuf[b, e, pl.ds(32 * j, 16)] = jnp.maximum(
                            g0 * x0 + b0, 0.0)
                        msgbuf[b, e, pl.ds(32 * j + 16, 16)] = jnp.maximum(
                            g1 * x1 + b1, 0.0)

                pltpu.async_copy(
                    msgbuf.at[b], accum.at[dstb.at[b]], ss[b], add=True)

                @pl.when(ci + 3 < NCHE)
                def _():
                    fire(ci + 3, b)

            return 0

        lax.fori_loop(0, NCHE // 3, body, 0)
        for b in range(NCHE - 3 * (NCHE // 3)):
            ci = 3 * (NCHE // 3) + b
            pltpu.make_async_copy(
                xl_hbm.at[pl.ds(0, CH)], xbuf.at[b], gx[b]).wait()
            pltpu.make_async_copy(
                gb_hbm.at[pl.ds(0, CH)], gbbuf.at[b], gg[b]).wait()
            for j in range(CH // 16):
                dstb[b, pl.ds(16 * j, 16)] = lax.shift_right_logical(
                    idst[pl.ds(ci * CH + 16 * j, 16)], 3)
            pltpu.make_async_copy(
                msgbuf.at[b], accum.at[pl.ds(0, CH)], ss[b]).wait()

            @plsc.parallel_loop(0, CH, unroll=2)
            def _(e):
                for j in range(DH // 32):
                    gv = plsc.bitcast(
                        gbbuf[b, e, pl.ds(DH // 2 + 16 * j, 16)], jnp.bfloat16)
                    bv = plsc.bitcast(
                        gbbuf[b, e, pl.ds(16 * j, 16)], jnp.bfloat16)
                    xv = plsc.bitcast(
                        xbuf[b, e, pl.ds(16 * j, 16)], jnp.bfloat16)
                    g0, g1 = plsc.unpack(
                        gv, format=plsc.PackFormat.INTERLEAVED,
                        preferred_element_type=jnp.float32)
                    b0, b1 = plsc.unpack(
                        bv, format=plsc.PackFormat.INTERLEAVED,
                        preferred_element_type=jnp.float32)
                    x0, x1 = plsc.unpack(
                        xv, format=plsc.PackFormat.INTERLEAVED,
                        preferred_element_type=jnp.float32)
                    msgbuf[b, e, pl.ds(32 * j, 16)] = jnp.maximum(
                        g0 * x0 + b0, 0.0)
                    msgbuf[b, e, pl.ds(32 * j + 16, 16)] = jnp.maximum(
                        g1 * x1 + b1, 0.0)

            pltpu.async_copy(
                msgbuf.at[b], accum.at[dstb.at[b]], ss[b], add=True)
        for b in range(3):
            pltpu.make_async_copy(
                msgbuf.at[b], accum.at[pl.ds(0, CH)], ss[b]).wait()
        plsc.subcore_barrier()
        pltpu.sync_copy(accum.at[pl.ds(rbase, RPT)],
                        out_hbm.at[pl.ds(rbase, RPT), pl.ds(c * DH, DH)])

    return k(xl_tab, gb_tab, srcr8, dstr8)


def _lohi(w64):
    lo = jnp.concatenate([w64[..., 0:16], w64[..., 32:48]], axis=-1)
    hi = jnp.concatenate([w64[..., 16:32], w64[..., 48:64]], axis=-1)
    return lo, hi


def kernel(x, edge_index, edge_type, mask, params):
    src = edge_index[0].astype(jnp.int32)
    dst = edge_index[1].astype(jnp.int32)
    rt = edge_type.astype(jnp.int32)
    srcr8 = src * (2 * R) + rt
    dstr8 = dst * (2 * R) + rt
    dstr4 = dst * R + rt

    cnt = _count_sc(dstr4)
    inv_exp = _prep_inv(cnt[0], cnt[1])
    iv = inv_exp.reshape(N, 4 * DH)
    inv_n = jnp.concatenate([iv, iv], axis=1)

    p = params
    h = _enc(x, p["enc"]["W"].T, p["enc"]["b"])

    gp = p["gru_s"]
    wa = jnp.concatenate(
        [gp["z0"]["W"].T, gp["r0"]["W"].T, gp["h0"]["W"].T], axis=1)
    wx = jnp.concatenate([gp["z1"]["W"].T, gp["r1"]["W"].T], axis=1)
    ws = jnp.concatenate(
        [gp["z2"]["W"].T, gp["r2"]["W"].T, gp["h2"]["W"].T], axis=1)
    wh1 = gp["h1"]["W"].T
    gbias = jnp.concatenate([
        gp["z0"]["b"] + gp["z1"]["b"] + gp["z2"]["b"],
        gp["r0"]["b"] + gp["r1"]["b"] + gp["r2"]["b"],
        gp["h0"]["b"] + gp["h1"]["b"] + gp["h2"]["b"],
    ]).reshape(1, 3 * D)

    for step in range(NSTEP):
        cp = p["conv"][step]
        xl_lo, xl_hi = [], []
        for c in range(NC):
            for r in range(R):
                lo, hi = _lohi(cp["lins"][r]["W"].T[:, c * DH:(c + 1) * DH])
                xl_lo.append(lo)
                xl_hi.append(hi)
        gb_lo, gb_hi, bias_lo, bias_hi = [], [], [], []
        for c in range(NC):
            for r in range(R):
                wt = cp["films"][r]["W"].T
                bb = cp["films"][r]["b"]
                for piece in (wt[:, c * DH:(c + 1) * DH],
                              wt[:, D + c * DH:D + (c + 1) * DH]):
                    lo, hi = _lohi(piece)
                    gb_lo.append(lo)
                    gb_hi.append(hi)
                for piece in (bb[c * DH:(c + 1) * DH],
                              bb[D + c * DH:D + (c + 1) * DH]):
                    lo, hi = _lohi(piece)
                    bias_lo.append(lo)
                    bias_hi.append(hi)
        w_cat = jnp.concatenate(
            [cp["lin_skip"]["W"].T, cp["film_skip"]["W"].T]
            + xl_lo + xl_hi + gb_lo + gb_hi,
            axis=1)
        bias_cat = jnp.concatenate(
            bias_lo + bias_hi).reshape(1, 8 * D)

        skip, xl, gb = _pre(h, w_cat, bias_cat, inv_n)
        xl_tab = xl.reshape(2 * NR, DH // 2)
        gb_tab = gb.reshape(2 * NR, DH)

        part = _edges_sc(xl_tab, gb_tab, srcr8, dstr8)
        q = _attn(h.reshape(B, L, D), mask)
        sn = jnp.repeat(q, L, axis=0)
        h = _gru(h, skip, part, sn, wa, wx, ws, wh1, gbias)

    return h.reshape(B, L, D)


# speedup vs baseline: 44.1255x; 1.1910x over previous
import functools
import math

import jax
import jax.numpy as jnp
from jax import lax
from jax.experimental import pallas as pl
from jax.experimental.pallas import tpu as pltpu
from jax.experimental.pallas import tpu_sc as plsc

N = 10000
B = 100
L = 100
D = 128
E = 320000
R = 4
NSTEP = 2
NR = N * R

NC = 2
NS = 16
DH = D // NC
EPT = E // NS
CH = 80
NCHE = EPT // CH
NW = NC * NS
EPW = E // NW
NCHC = EPW // CH
RPT = N // NS
CRP = NR // NS


def _enc_body(x_ref, w_ref, b_ref, o_ref):
    y = jnp.dot(x_ref[...], w_ref[...], preferred_element_type=jnp.float32)
    o_ref[...] = jnp.maximum(y + b_ref[...], 0.0)


def _enc(x, w_t, b):
    blk = 2000
    return pl.pallas_call(
        _enc_body,
        grid=(N // blk,),
        in_specs=[
            pl.BlockSpec((blk, D), lambda i: (i, 0)),
            pl.BlockSpec((D, D), lambda i: (0, 0)),
            pl.BlockSpec((1, D), lambda i: (0, 0)),
        ],
        out_specs=pl.BlockSpec((blk, D), lambda i: (i, 0)),
        out_shape=jax.ShapeDtypeStruct((N, D), jnp.float32),
    )(x, w_t, b.reshape(1, D))


def _inv_body(c_ref, o_ref):
    tiles = []
    for r in range(R):
        s = c_ref[0, :, 16 * r:16 * r + 16] + c_ref[1, :, 16 * r:16 * r + 16]
        inv = 1.0 / jnp.maximum(s, 1.0)
        tiles.append(jnp.concatenate([inv] * 4, axis=1))
    o_ref[...] = jnp.concatenate(tiles + tiles, axis=1)


def _prep_inv(cnt2):
    blk = 2000
    return pl.pallas_call(
        _inv_body,
        grid=(N // blk,),
        in_specs=[pl.BlockSpec((NC, blk, D), lambda i: (0, i, 0))],
        out_specs=pl.BlockSpec((blk, 4 * D), lambda i: (i, 0)),
        out_shape=jax.ShapeDtypeStruct((N, 4 * D), jnp.float32),
    )(cnt2)


def _pack_pair(lo, hi):
    lo_b = lax.bitcast_convert_type(lo.astype(jnp.bfloat16), jnp.uint16)
    hi_b = lax.bitcast_convert_type(hi.astype(jnp.bfloat16), jnp.uint16)
    u = lo_b.astype(jnp.uint32) | (hi_b.astype(jnp.uint32) << 16)
    return lax.bitcast_convert_type(u, jnp.float32)


def _pre_body(h_ref, w_ref, bias_ref, inv_ref, skip_ref, xl_ref, gb_ref):
    y = jnp.dot(h_ref[...], w_ref[...], preferred_element_type=jnp.float32)
    skip_ref[...] = jnp.maximum(y[:, 256:384] * y[:, 0:128] + y[:, 128:256], 0.0)
    xlp = _pack_pair(y[:, 384:640], y[:, 640:896])
    xl_ref[:, 0, :] = xlp[:, 0:128]
    xl_ref[:, 1, :] = xlp[:, 128:256]
    inv = inv_ref[...]
    bias = bias_ref[...]
    glo = (y[:, 896:1408] + bias[:, 0:512]) * inv
    ghi = (y[:, 1408:1920] + bias[:, 512:1024]) * inv
    gbp = _pack_pair(glo, ghi)
    gb_ref[:, 0, :] = gbp[:, 0:128]
    gb_ref[:, 1, :] = gbp[:, 128:256]
    gb_ref[:, 2, :] = gbp[:, 256:384]
    gb_ref[:, 3, :] = gbp[:, 384:512]


def _pre(h, w_cat, bias_cat, inv_n):
    blk = 2000
    return pl.pallas_call(
        _pre_body,
        grid=(N // blk,),
        in_specs=[
            pl.BlockSpec((blk, D), lambda i: (i, 0)),
            pl.BlockSpec((D, 15 * D), lambda i: (0, 0)),
            pl.BlockSpec((1, 8 * D), lambda i: (0, 0)),
            pl.BlockSpec((blk, 4 * D), lambda i: (i, 0)),
        ],
        out_specs=[
            pl.BlockSpec((blk, D), lambda i: (i, 0)),
            pl.BlockSpec((blk, 2, D), lambda i: (i, 0, 0)),
            pl.BlockSpec((blk, 4, D), lambda i: (i, 0, 0)),
        ],
        out_shape=[
            jax.ShapeDtypeStruct((N, D), jnp.float32),
            jax.ShapeDtypeStruct((N, 2, D), jnp.float32),
            jax.ShapeDtypeStruct((N, 4, D), jnp.float32),
        ],
    )(h, w_cat, bias_cat, inv_n)


_AG = 10


def _attn_body(h_ref, m_ref, q_ref):
    for g in range(_AG):
        hm = h_ref[g] * m_ref[g]
        logits = lax.dot_general(
            hm, hm, (((1,), (1,)), ((), ())),
            preferred_element_type=jnp.float32) * (1.0 / math.sqrt(D))
        mx = jnp.max(logits, axis=-1, keepdims=True)
        e = jnp.exp(logits - mx)
        aw = e / jnp.sum(e, axis=-1, keepdims=True)
        a = jnp.dot(aw, hm, preferred_element_type=jnp.float32)
        q_ref[g, 0] = jnp.mean(a, axis=0)


def _attn(h3, mask):
    out = pl.pallas_call(
        _attn_body,
        grid=(B // _AG,),
        in_specs=[
            pl.BlockSpec((_AG, L, D), lambda i: (i, 0, 0)),
            pl.BlockSpec((_AG, L, 1), lambda i: (i, 0, 0)),
        ],
        out_specs=pl.BlockSpec((_AG, 1, D), lambda i: (i, 0, 0)),
        out_shape=jax.ShapeDtypeStruct((B, 1, D), jnp.float32),
    )(h3, mask)
    return out.reshape(B, D)


def _gru_body(h_ref, skip_ref, p_ref, sn_ref, wa_ref, wx_ref,
              ws_ref, wh1_ref, bias_ref, o_ref):
    h = h_ref[...]
    a = skip_ref[...] + p_ref[...]
    sn = sn_ref[...]
    ya = jnp.dot(a, wa_ref[...], preferred_element_type=jnp.float32)
    yx = jnp.dot(h, wx_ref[...], preferred_element_type=jnp.float32)
    ys = jnp.dot(sn, ws_ref[...], preferred_element_type=jnp.float32)
    bias = bias_ref[...]
    z = jax.nn.sigmoid(ya[:, 0:D] + yx[:, 0:D] + ys[:, 0:D] + bias[:, 0:D])
    r = jax.nn.sigmoid(
        ya[:, D:2 * D] + yx[:, D:2 * D] + ys[:, D:2 * D] + bias[:, D:2 * D])
    hr = jnp.dot(h * r, wh1_ref[...], preferred_element_type=jnp.float32)
    hh = jnp.maximum(
        ya[:, 2 * D:3 * D] + hr + ys[:, 2 * D:3 * D] + bias[:, 2 * D:3 * D], 0.0)
    o_ref[...] = hh * z + h * (1.0 - z)


def _gru(h, skip, p, sn, wa, wx, ws, wh1, bias):
    blk = 2000
    return pl.pallas_call(
        _gru_body,
        grid=(N // blk,),
        in_specs=[
            pl.BlockSpec((blk, D), lambda i: (i, 0)),
            pl.BlockSpec((blk, D), lambda i: (i, 0)),
            pl.BlockSpec((blk, D), lambda i: (i, 0)),
            pl.BlockSpec((blk, D), lambda i: (i, 0)),
            pl.BlockSpec((D, 3 * D), lambda i: (0, 0)),
            pl.BlockSpec((D, 2 * D), lambda i: (0, 0)),
            pl.BlockSpec((D, 3 * D), lambda i: (0, 0)),
            pl.BlockSpec((D, D), lambda i: (0, 0)),
            pl.BlockSpec((1, 3 * D), lambda i: (0, 0)),
        ],
        out_specs=pl.BlockSpec((blk, D), lambda i: (i, 0)),
        out_shape=jax.ShapeDtypeStruct((N, D), jnp.float32),
    )(h, skip, p, sn, wa, wx, ws, wh1, bias)


def _count_sc(dstr8):
    mesh = plsc.VectorSubcoreMesh(core_axis_name="c", subcore_axis_name="s")

    @functools.partial(
        pl.kernel,
        out_type=jax.ShapeDtypeStruct((NC, 8 * N, 16), jnp.float32),
        mesh=mesh,
        compiler_params=pltpu.CompilerParams(use_tc_tiling_on_sc=False),
        scratch_types=[
            pltpu.VMEM((EPW,), jnp.int32),
            pltpu.VMEM((NCHC, CH), jnp.int32),
            pltpu.VMEM((CH, 16), jnp.float32),
            pltpu.VMEM((100, 16), jnp.float32),
            pltpu.VMEM_SHARED((8 * N, 16), jnp.float32),
            pltpu.SemaphoreType.DMA,
        ],
    )
    def k(dstr_hbm, out_hbm, dbuf, ibuf2d, ones_v, zbuf, cnt_sh, sem):
        c = lax.axis_index("c")
        s = lax.axis_index("s")
        wid = s * NC + c
        unit = jnp.full((16,), 1.0, jnp.float32)
        zero = jnp.zeros((16,), jnp.float32)

        pltpu.sync_copy(dstr_hbm.at[pl.ds(wid * EPW, EPW)], dbuf)

        def mkidx(ci, _):
            for j in range(CH // 16):
                ibuf2d[ci, pl.ds(16 * j, 16)] = dbuf[pl.ds(ci * CH + 16 * j, 16)]
            return 0

        lax.fori_loop(0, NCHC, mkidx, 0)

        def fill(i, _):
            ones_v[i, :] = unit
            return 0

        lax.fori_loop(0, CH, fill, 0)

        def fillz(i, _):
            zbuf[i, :] = zero
            return 0

        lax.fori_loop(0, 100, fillz, 0)

        crp = 8 * N // NS

        def zcp(i, _):
            pltpu.sync_copy(zbuf, cnt_sh.at[pl.ds(s * crp + i * 100, 100)])
            return 0

        lax.fori_loop(0, crp // 100, zcp, 0)
        plsc.subcore_barrier()

        def chunk(i, _):
            pltpu.async_copy(ones_v, cnt_sh.at[ibuf2d.at[i]], sem, add=True)

            @pl.when(i >= 4)
            def _():
                pltpu.make_async_copy(
                    ones_v, cnt_sh.at[pl.ds(0, CH)], sem).wait()

            return 0

        lax.fori_loop(0, NCHC, chunk, 0)
        for _ in range(4):
            pltpu.make_async_copy(ones_v, cnt_sh.at[pl.ds(0, CH)], sem).wait()
        plsc.subcore_barrier()
        sl = pl.ds(s * crp, crp)
        pltpu.sync_copy(cnt_sh.at[sl], out_hbm.at[c, sl])

    return k(dstr8)


def _edges_sc(xl_tab, gb_tab, srcr8, dstr8):
    mesh = plsc.VectorSubcoreMesh(core_axis_name="c", subcore_axis_name="s")

    @functools.partial(
        pl.kernel,
        out_type=jax.ShapeDtypeStruct((N, D), jnp.float32),
        mesh=mesh,
        compiler_params=pltpu.CompilerParams(
            use_tc_tiling_on_sc=False, needs_layout_passes=False),
        scratch_types=[
            pltpu.VMEM((EPT,), jnp.int32),
            pltpu.VMEM((EPT,), jnp.int32),
            pltpu.VMEM((3, CH), jnp.int32),
            pltpu.VMEM((3, CH, DH // 2), jnp.float32),
            pltpu.VMEM((3, CH, DH), jnp.float32),
            pltpu.VMEM((3, CH, DH), jnp.float32),
            pltpu.VMEM_SHARED((N, DH), jnp.float32),
            pltpu.SemaphoreType.DMA,
            pltpu.SemaphoreType.DMA,
            pltpu.SemaphoreType.DMA,
            pltpu.SemaphoreType.DMA,
            pltpu.SemaphoreType.DMA,
            pltpu.SemaphoreType.DMA,
            pltpu.SemaphoreType.DMA,
            pltpu.SemaphoreType.DMA,
            pltpu.SemaphoreType.DMA,
        ],
    )
    def k(xl_hbm, gb_hbm, srcr_hbm, dstr_hbm, out_hbm,
          isrc, idst, dstb, xbuf, gbbuf, msgbuf, accum,
          gx0, gx1, gx2, gg0, gg1, gg2, ss0, ss1, ss2):
        gx = (gx0, gx1, gx2)
        gg = (gg0, gg1, gg2)
        ss = (ss0, ss1, ss2)
        c = lax.axis_index("c")
        s = lax.axis_index("s")
        ebase = s * EPT
        zero = jnp.zeros((16,), jnp.float32)

        pltpu.sync_copy(srcr_hbm.at[pl.ds(ebase, EPT)], isrc)
        pltpu.sync_copy(dstr_hbm.at[pl.ds(ebase, EPT)], idst)
        off = c * R

        def adj(i, _):
            sl = pl.ds(16 * i, 16)
            isrc[sl] = isrc[sl] + off
            idst[sl] = idst[sl] + off
            return 0

        lax.fori_loop(0, EPT // 16, adj, 0)

        def zb(i, _):
            for j in range(DH // 16):
                msgbuf[0, i, pl.ds(j * 16, 16)] = zero
            return 0

        lax.fori_loop(0, CH, zb, 0)
        rbase = s * RPT
        for i in range(RPT // CH):
            pltpu.sync_copy(msgbuf.at[0], accum.at[pl.ds(rbase + i * CH, CH)])
        rem = RPT - (RPT // CH) * CH
        pltpu.sync_copy(msgbuf.at[0, pl.ds(0, rem)],
                        accum.at[pl.ds(rbase + RPT - rem, rem)])
        plsc.subcore_barrier()

        def fire(ci, b):
            pltpu.async_copy(
                xl_hbm.at[isrc.at[pl.ds(ci * CH, CH)]], xbuf.at[b], gx[b])
            pltpu.async_copy(
                gb_hbm.at[idst.at[pl.ds(ci * CH, CH)]], gbbuf.at[b], gg[b])

        fire(0, 0)
        fire(1, 1)
        fire(2, 2)

        def body(i, _):
            for b in range(3):
                ci = 3 * i + b
                pltpu.make_async_copy(
                    xl_hbm.at[pl.ds(0, CH)], xbuf.at[b], gx[b]).wait()
                pltpu.make_async_copy(
                    gb_hbm.at[pl.ds(0, CH)], gbbuf.at[b], gg[b]).wait()

                for j in range(CH // 16):
                    dstb[b, pl.ds(16 * j, 16)] = lax.shift_right_logical(
                        idst[pl.ds(ci * CH + 16 * j, 16)], 3)

                @pl.when(ci >= 3)
                def _():
                    pltpu.make_async_copy(
                        msgbuf.at[b], accum.at[pl.ds(0, CH)], ss[b]).wait()

                @plsc.parallel_loop(0, CH, unroll=2)
                def _(e):
                    for j in range(DH // 32):
                        gv = plsc.bitcast(
                            gbbuf[b, e, pl.ds(DH // 2 + 16 * j, 16)],
                            jnp.bfloat16)
                        bv = plsc.bitcast(
                            gbbuf[b, e, pl.ds(16 * j, 16)], jnp.bfloat16)
                        xv = plsc.bitcast(
                            xbuf[b, e, pl.ds(16 * j, 16)], jnp.bfloat16)
                        g0, g1 = plsc.unpack(
                            gv, format=plsc.PackFormat.INTERLEAVED,
                            preferred_element_type=jnp.float32)
                        b0, b1 = plsc.unpack(
                            bv, format=plsc.PackFormat.INTERLEAVED,
                            preferred_element_type=jnp.float32)
                        x0, x1 = plsc.unpack(
                            xv, format=plsc.PackFormat.INTERLEAVED,
                            preferred_element_type=jnp.float32)
                        msgbuf[b, e, pl.ds(32 * j, 16)] = jnp.maximum(
                            g0 * x0 + b0, 0.0)
                        msgbuf[b, e, pl.ds(32 * j + 16, 16)] = jnp.maximum(
                            g1 * x1 + b1, 0.0)

                pltpu.async_copy(
                    msgbuf.at[b], accum.at[dstb.at[b]], ss[b], add=True)

                @pl.when(ci + 3 < NCHE)
                def _():
                    fire(ci + 3, b)

            return 0

        lax.fori_loop(0, NCHE // 3, body, 0)
        for b in range(NCHE - 3 * (NCHE // 3)):
            ci = 3 * (NCHE // 3) + b
            pltpu.make_async_copy(
                xl_hbm.at[pl.ds(0, CH)], xbuf.at[b], gx[b]).wait()
            pltpu.make_async_copy(
                gb_hbm.at[pl.ds(0, CH)], gbbuf.at[b], gg[b]).wait()
            for j in range(CH // 16):
                dstb[b, pl.ds(16 * j, 16)] = lax.shift_right_logical(
                    idst[pl.ds(ci * CH + 16 * j, 16)], 3)
            pltpu.make_async_copy(
                msgbuf.at[b], accum.at[pl.ds(0, CH)], ss[b]).wait()

            @plsc.parallel_loop(0, CH, unroll=2)
            def _(e):
                for j in range(DH // 32):
                    gv = plsc.bitcast(
                        gbbuf[b, e, pl.ds(DH // 2 + 16 * j, 16)], jnp.bfloat16)
                    bv = plsc.bitcast(
                        gbbuf[b, e, pl.ds(16 * j, 16)], jnp.bfloat16)
                    xv = plsc.bitcast(
                        xbuf[b, e, pl.ds(16 * j, 16)], jnp.bfloat16)
                    g0, g1 = plsc.unpack(
                        gv, format=plsc.PackFormat.INTERLEAVED,
                        preferred_element_type=jnp.float32)
                    b0, b1 = plsc.unpack(
                        bv, format=plsc.PackFormat.INTERLEAVED,
                        preferred_element_type=jnp.float32)
                    x0, x1 = plsc.unpack(
                        xv, format=plsc.PackFormat.INTERLEAVED,
                        preferred_element_type=jnp.float32)
                    msgbuf[b, e, pl.ds(32 * j, 16)] = jnp.maximum(
                        g0 * x0 + b0, 0.0)
                    msgbuf[b, e, pl.ds(32 * j + 16, 16)] = jnp.maximum(
                        g1 * x1 + b1, 0.0)

            pltpu.async_copy(
                msgbuf.at[b], accum.at[dstb.at[b]], ss[b], add=True)
        for b in range(3):
            pltpu.make_async_copy(
                msgbuf.at[b], accum.at[pl.ds(0, CH)], ss[b]).wait()
        plsc.subcore_barrier()
        pltpu.sync_copy(accum.at[pl.ds(rbase, RPT)],
                        out_hbm.at[pl.ds(rbase, RPT), pl.ds(c * DH, DH)])

    return k(xl_tab, gb_tab, srcr8, dstr8)


def _lohi(w64):
    lo = jnp.concatenate([w64[..., 0:16], w64[..., 32:48]], axis=-1)
    hi = jnp.concatenate([w64[..., 16:32], w64[..., 48:64]], axis=-1)
    return lo, hi


def kernel(x, edge_index, edge_type, mask, params):
    src = edge_index[0].astype(jnp.int32)
    dst = edge_index[1].astype(jnp.int32)
    rt = edge_type.astype(jnp.int32)
    srcr8 = src * (2 * R) + rt
    dstr8 = dst * (2 * R) + rt

    cnt = _count_sc(dstr8)
    cnt2 = cnt.reshape(NC, N, D)
    inv_n = _prep_inv(cnt2)

    p = params
    h = _enc(x, p["enc"]["W"].T, p["enc"]["b"])

    gp = p["gru_s"]
    wa = jnp.concatenate(
        [gp["z0"]["W"].T, gp["r0"]["W"].T, gp["h0"]["W"].T], axis=1)
    wx = jnp.concatenate([gp["z1"]["W"].T, gp["r1"]["W"].T], axis=1)
    ws = jnp.concatenate(
        [gp["z2"]["W"].T, gp["r2"]["W"].T, gp["h2"]["W"].T], axis=1)
    wh1 = gp["h1"]["W"].T
    gbias = jnp.concatenate([
        gp["z0"]["b"] + gp["z1"]["b"] + gp["z2"]["b"],
        gp["r0"]["b"] + gp["r1"]["b"] + gp["r2"]["b"],
        gp["h0"]["b"] + gp["h1"]["b"] + gp["h2"]["b"],
    ]).reshape(1, 3 * D)

    for step in range(NSTEP):
        cp = p["conv"][step]
        xl_lo, xl_hi = [], []
        for c in range(NC):
            for r in range(R):
                lo, hi = _lohi(cp["lins"][r]["W"].T[:, c * DH:(c + 1) * DH])
                xl_lo.append(lo)
                xl_hi.append(hi)
        gb_lo, gb_hi, bias_lo, bias_hi = [], [], [], []
        for c in range(NC):
            for r in range(R):
                wt = cp["films"][r]["W"].T
                bb = cp["films"][r]["b"]
                for piece in (wt[:, c * DH:(c + 1) * DH],
                              wt[:, D + c * DH:D + (c + 1) * DH]):
                    lo, hi = _lohi(piece)
                    gb_lo.append(lo)
                    gb_hi.append(hi)
                for piece in (bb[c * DH:(c + 1) * DH],
                              bb[D + c * DH:D + (c + 1) * DH]):
                    lo, hi = _lohi(piece)
                    bias_lo.append(lo)
                    bias_hi.append(hi)
        w_cat = jnp.concatenate(
            [cp["lin_skip"]["W"].T, cp["film_skip"]["W"].T]
            + xl_lo + xl_hi + gb_lo + gb_hi,
            axis=1)
        bias_cat = jnp.concatenate(
            bias_lo + bias_hi).reshape(1, 8 * D)

        skip, xl, gb = _pre(h, w_cat, bias_cat, inv_n)
        xl_tab = xl.reshape(2 * NR, DH // 2)
        gb_tab = gb.reshape(2 * NR, DH)

        part = _edges_sc(xl_tab, gb_tab, srcr8, dstr8)
        q = _attn(h.reshape(B, L, D), mask)
        sn = jnp.repeat(q, L, axis=0)
        h = _gru(h, skip, part, sn, wa, wx, ws, wh1, gbias)

    return h.reshape(B, L, D)


# speedup vs baseline: 44.5413x; 1.0094x over previous
import functools
import math

import jax
import jax.numpy as jnp
from jax import lax
from jax.experimental import pallas as pl
from jax.experimental.pallas import tpu as pltpu
from jax.experimental.pallas import tpu_sc as plsc

N = 10000
B = 100
L = 100
D = 128
E = 320000
R = 4
NSTEP = 2
NR = N * R

NC = 2
NS = 16
DH = D // NC
EPT = E // NS
CH = 80
NCHE = EPT // CH
NW = NC * NS
EPW = E // NW
NCHC = EPW // CH
RPT = N // NS
CRP = NR // NS


def _enc_body(x_ref, w_ref, b_ref, o_ref):
    y = jnp.dot(x_ref[...], w_ref[...], preferred_element_type=jnp.float32)
    o_ref[...] = jnp.maximum(y + b_ref[...], 0.0)


def _enc(x, w_t, b):
    blk = 2000
    return pl.pallas_call(
        _enc_body,
        grid=(N // blk,),
        in_specs=[
            pl.BlockSpec((blk, D), lambda i: (i, 0)),
            pl.BlockSpec((D, D), lambda i: (0, 0)),
            pl.BlockSpec((1, D), lambda i: (0, 0)),
        ],
        out_specs=pl.BlockSpec((blk, D), lambda i: (i, 0)),
        out_shape=jax.ShapeDtypeStruct((N, D), jnp.float32),
    )(x, w_t, b.reshape(1, D))


def _inv_body(c_ref, o_ref):
    cols = []
    for r in range(R):
        s = c_ref[0, :, 16 * r:16 * r + 16] + c_ref[1, :, 16 * r:16 * r + 16]
        inv = 1.0 / jnp.maximum(s, 1.0)
        cols.append(inv[:, 0:1])
    o_ref[...] = jnp.concatenate(cols + cols, axis=1)


def _prep_inv(cnt2):
    blk = 2000
    return pl.pallas_call(
        _inv_body,
        grid=(N // blk,),
        in_specs=[pl.BlockSpec((NC, blk, D), lambda i: (0, i, 0))],
        out_specs=pl.BlockSpec((blk, 8), lambda i: (i, 0)),
        out_shape=jax.ShapeDtypeStruct((N, 8), jnp.float32),
    )(cnt2)


def _pack_pair(lo, hi):
    lo_b = lax.bitcast_convert_type(lo.astype(jnp.bfloat16), jnp.uint16)
    hi_b = lax.bitcast_convert_type(hi.astype(jnp.bfloat16), jnp.uint16)
    u = lo_b.astype(jnp.uint32) | (hi_b.astype(jnp.uint32) << 16)
    return lax.bitcast_convert_type(u, jnp.float32)


def _pre_body(h_ref, w_ref, bias_ref, inv_ref, skip_ref, xl_ref, gb_ref):
    y = jnp.dot(h_ref[...], w_ref[...], preferred_element_type=jnp.float32)
    skip_ref[...] = jnp.maximum(y[:, 256:384] * y[:, 0:128] + y[:, 128:256], 0.0)
    xlp = _pack_pair(y[:, 384:640], y[:, 640:896])
    xl_ref[:, 0, :] = xlp[:, 0:128]
    xl_ref[:, 1, :] = xlp[:, 128:256]
    nb = y.shape[0]
    iv = inv_ref[...]
    inv = jnp.concatenate(
        [jnp.broadcast_to(iv[:, r:r + 1], (nb, 64)) for r in range(R)],
        axis=1)
    inv = jnp.concatenate([inv, inv], axis=1)
    bias = bias_ref[...]
    glo = (y[:, 896:1408] + bias[:, 0:512]) * inv
    ghi = (y[:, 1408:1920] + bias[:, 512:1024]) * inv
    gbp = _pack_pair(glo, ghi)
    gb_ref[:, 0, :] = gbp[:, 0:128]
    gb_ref[:, 1, :] = gbp[:, 128:256]
    gb_ref[:, 2, :] = gbp[:, 256:384]
    gb_ref[:, 3, :] = gbp[:, 384:512]


def _pre(h, w_cat, bias_cat, inv_n):
    blk = 2000
    return pl.pallas_call(
        _pre_body,
        grid=(N // blk,),
        in_specs=[
            pl.BlockSpec((blk, D), lambda i: (i, 0)),
            pl.BlockSpec((D, 15 * D), lambda i: (0, 0)),
            pl.BlockSpec((1, 8 * D), lambda i: (0, 0)),
            pl.BlockSpec((blk, 8), lambda i: (i, 0)),
        ],
        out_specs=[
            pl.BlockSpec((blk, D), lambda i: (i, 0)),
            pl.BlockSpec((blk, 2, D), lambda i: (i, 0, 0)),
            pl.BlockSpec((blk, 4, D), lambda i: (i, 0, 0)),
        ],
        out_shape=[
            jax.ShapeDtypeStruct((N, D), jnp.float32),
            jax.ShapeDtypeStruct((N, 2, D), jnp.float32),
            jax.ShapeDtypeStruct((N, 4, D), jnp.float32),
        ],
    )(h, w_cat, bias_cat, inv_n)


_AG = 10


def _attn_body(h_ref, m_ref, q_ref):
    for g in range(_AG):
        hm = h_ref[g] * m_ref[g]
        logits = lax.dot_general(
            hm, hm, (((1,), (1,)), ((), ())),
            preferred_element_type=jnp.float32) * (1.0 / math.sqrt(D))
        mx = jnp.max(logits, axis=-1, keepdims=True)
        e = jnp.exp(logits - mx)
        aw = e / jnp.sum(e, axis=-1, keepdims=True)
        a = jnp.dot(aw, hm, preferred_element_type=jnp.float32)
        q_ref[g, 0] = jnp.mean(a, axis=0)


def _attn(h3, mask):
    out = pl.pallas_call(
        _attn_body,
        grid=(B // _AG,),
        in_specs=[
            pl.BlockSpec((_AG, L, D), lambda i: (i, 0, 0)),
            pl.BlockSpec((_AG, L, 1), lambda i: (i, 0, 0)),
        ],
        out_specs=pl.BlockSpec((_AG, 1, D), lambda i: (i, 0, 0)),
        out_shape=jax.ShapeDtypeStruct((B, 1, D), jnp.float32),
    )(h3, mask)
    return out.reshape(B, D)


def _gru_body(h_ref, skip_ref, p_ref, sn_ref, wa_ref, wx_ref,
              ws_ref, wh1_ref, bias_ref, o_ref):
    h = h_ref[...]
    a = skip_ref[...] + p_ref[...]
    sn = sn_ref[...]
    ya = jnp.dot(a, wa_ref[...], preferred_element_type=jnp.float32)
    yx = jnp.dot(h, wx_ref[...], preferred_element_type=jnp.float32)
    ys = jnp.dot(sn, ws_ref[...], preferred_element_type=jnp.float32)
    bias = bias_ref[...]
    z = jax.nn.sigmoid(ya[:, 0:D] + yx[:, 0:D] + ys[:, 0:D] + bias[:, 0:D])
    r = jax.nn.sigmoid(
        ya[:, D:2 * D] + yx[:, D:2 * D] + ys[:, D:2 * D] + bias[:, D:2 * D])
    hr = jnp.dot(h * r, wh1_ref[...], preferred_element_type=jnp.float32)
    hh = jnp.maximum(
        ya[:, 2 * D:3 * D] + hr + ys[:, 2 * D:3 * D] + bias[:, 2 * D:3 * D], 0.0)
    o_ref[...] = hh * z + h * (1.0 - z)


def _gru(h, skip, p, sn, wa, wx, ws, wh1, bias):
    blk = 2000
    return pl.pallas_call(
        _gru_body,
        grid=(N // blk,),
        in_specs=[
            pl.BlockSpec((blk, D), lambda i: (i, 0)),
            pl.BlockSpec((blk, D), lambda i: (i, 0)),
            pl.BlockSpec((blk, D), lambda i: (i, 0)),
            pl.BlockSpec((blk, D), lambda i: (i, 0)),
            pl.BlockSpec((D, 3 * D), lambda i: (0, 0)),
            pl.BlockSpec((D, 2 * D), lambda i: (0, 0)),
            pl.BlockSpec((D, 3 * D), lambda i: (0, 0)),
            pl.BlockSpec((D, D), lambda i: (0, 0)),
            pl.BlockSpec((1, 3 * D), lambda i: (0, 0)),
        ],
        out_specs=pl.BlockSpec((blk, D), lambda i: (i, 0)),
        out_shape=jax.ShapeDtypeStruct((N, D), jnp.float32),
    )(h, skip, p, sn, wa, wx, ws, wh1, bias)


def _count_sc(dstr8):
    mesh = plsc.VectorSubcoreMesh(core_axis_name="c", subcore_axis_name="s")

    @functools.partial(
        pl.kernel,
        out_type=jax.ShapeDtypeStruct((NC, 8 * N, 16), jnp.float32),
        mesh=mesh,
        compiler_params=pltpu.CompilerParams(use_tc_tiling_on_sc=False),
        scratch_types=[
            pltpu.VMEM((EPW,), jnp.int32),
            pltpu.VMEM((NCHC, CH), jnp.int32),
            pltpu.VMEM((CH, 16), jnp.float32),
            pltpu.VMEM((100, 16), jnp.float32),
            pltpu.VMEM_SHARED((8 * N, 16), jnp.float32),
            pltpu.SemaphoreType.DMA,
        ],
    )
    def k(dstr_hbm, out_hbm, dbuf, ibuf2d, ones_v, zbuf, cnt_sh, sem):
        c = lax.axis_index("c")
        s = lax.axis_index("s")
        wid = s * NC + c
        unit = jnp.full((16,), 1.0, jnp.float32)
        zero = jnp.zeros((16,), jnp.float32)

        pltpu.sync_copy(dstr_hbm.at[pl.ds(wid * EPW, EPW)], dbuf)

        def mkidx(ci, _):
            for j in range(CH // 16):
                ibuf2d[ci, pl.ds(16 * j, 16)] = dbuf[pl.ds(ci * CH + 16 * j, 16)]
            return 0

        lax.fori_loop(0, NCHC, mkidx, 0)

        def fill(i, _):
            ones_v[i, :] = unit
            return 0

        lax.fori_loop(0, CH, fill, 0)

        def fillz(i, _):
            zbuf[i, :] = zero
            return 0

        lax.fori_loop(0, 100, fillz, 0)

        crp = 8 * N // NS

        def zcp(i, _):
            pltpu.sync_copy(zbuf, cnt_sh.at[pl.ds(s * crp + i * 100, 100)])
            return 0

        lax.fori_loop(0, crp // 100, zcp, 0)
        plsc.subcore_barrier()

        def chunk(i, _):
            pltpu.async_copy(ones_v, cnt_sh.at[ibuf2d.at[i]], sem, add=True)

            @pl.when(i >= 4)
            def _():
                pltpu.make_async_copy(
                    ones_v, cnt_sh.at[pl.ds(0, CH)], sem).wait()

            return 0

        lax.fori_loop(0, NCHC, chunk, 0)
        for _ in range(4):
            pltpu.make_async_copy(ones_v, cnt_sh.at[pl.ds(0, CH)], sem).wait()
        plsc.subcore_barrier()
        sl = pl.ds(s * crp, crp)
        pltpu.sync_copy(cnt_sh.at[sl], out_hbm.at[c, sl])

    return k(dstr8)


def _edges_sc(xl_tab, gb_tab, srcr8, dstr8):
    mesh = plsc.VectorSubcoreMesh(core_axis_name="c", subcore_axis_name="s")

    @functools.partial(
        pl.kernel,
        out_type=jax.ShapeDtypeStruct((N, D), jnp.float32),
        mesh=mesh,
        compiler_params=pltpu.CompilerParams(
            use_tc_tiling_on_sc=False, needs_layout_passes=False),
        scratch_types=[
            pltpu.VMEM((EPT,), jnp.int32),
            pltpu.VMEM((EPT,), jnp.int32),
            pltpu.VMEM((3, CH), jnp.int32),
            pltpu.VMEM((3, CH, DH // 2), jnp.float32),
            pltpu.VMEM((3, CH, DH), jnp.float32),
            pltpu.VMEM((3, CH, DH), jnp.float32),
            pltpu.VMEM_SHARED((N, DH), jnp.float32),
            pltpu.SemaphoreType.DMA,
            pltpu.SemaphoreType.DMA,
            pltpu.SemaphoreType.DMA,
            pltpu.SemaphoreType.DMA,
            pltpu.SemaphoreType.DMA,
            pltpu.SemaphoreType.DMA,
            pltpu.SemaphoreType.DMA,
            pltpu.SemaphoreType.DMA,
            pltpu.SemaphoreType.DMA,
        ],
    )
    def k(xl_hbm, gb_hbm, srcr_hbm, dstr_hbm, out_hbm,
          isrc, idst, dstb, xbuf, gbbuf, msgbuf, accum,
          gx0, gx1, gx2, gg0, gg1, gg2, ss0, ss1, ss2):
        gx = (gx0, gx1, gx2)
        gg = (gg0, gg1, gg2)
        ss = (ss0, ss1, ss2)
        c = lax.axis_index("c")
        s = lax.axis_index("s")
        ebase = s * EPT
        zero = jnp.zeros((16,), jnp.float32)

        pltpu.sync_copy(srcr_hbm.at[pl.ds(ebase, EPT)], isrc)
        pltpu.sync_copy(dstr_hbm.at[pl.ds(ebase, EPT)], idst)
        off = c * R

        def adj(i, _):
            sl = pl.ds(16 * i, 16)
            isrc[sl] = isrc[sl] + off
            idst[sl] = idst[sl] + off
            return 0

        lax.fori_loop(0, EPT // 16, adj, 0)

        def zb(i, _):
            for j in range(DH // 16):
                msgbuf[0, i, pl.ds(j * 16, 16)] = zero
            return 0

        lax.fori_loop(0, CH, zb, 0)
        rbase = s * RPT
        for i in range(RPT // CH):
            pltpu.sync_copy(msgbuf.at[0], accum.at[pl.ds(rbase + i * CH, CH)])
        rem = RPT - (RPT // CH) * CH
        pltpu.sync_copy(msgbuf.at[0, pl.ds(0, rem)],
                        accum.at[pl.ds(rbase + RPT - rem, rem)])
        plsc.subcore_barrier()

        def fire(ci, b):
            pltpu.async_copy(
                xl_hbm.at[isrc.at[pl.ds(ci * CH, CH)]], xbuf.at[b], gx[b])
            pltpu.async_copy(
                gb_hbm.at[idst.at[pl.ds(ci * CH, CH)]], gbbuf.at[b], gg[b])

        fire(0, 0)
        fire(1, 1)
        fire(2, 2)

        def body(i, _):
            for b in range(3):
                ci = 3 * i + b
                pltpu.make_async_copy(
                    xl_hbm.at[pl.ds(0, CH)], xbuf.at[b], gx[b]).wait()
                pltpu.make_async_copy(
                    gb_hbm.at[pl.ds(0, CH)], gbbuf.at[b], gg[b]).wait()

                for j in range(CH // 16):
                    dstb[b, pl.ds(16 * j, 16)] = lax.shift_right_logical(
                        idst[pl.ds(ci * CH + 16 * j, 16)], 3)

                @pl.when(ci >= 3)
                def _():
                    pltpu.make_async_copy(
                        msgbuf.at[b], accum.at[pl.ds(0, CH)], ss[b]).wait()

                @plsc.parallel_loop(0, CH, unroll=2)
                def _(e):
                    for j in range(DH // 32):
                        gv = plsc.bitcast(
                            gbbuf[b, e, pl.ds(DH // 2 + 16 * j, 16)],
                            jnp.bfloat16)
                        bv = plsc.bitcast(
                            gbbuf[b, e, pl.ds(16 * j, 16)], jnp.bfloat16)
                        xv = plsc.bitcast(
                            xbuf[b, e, pl.ds(16 * j, 16)], jnp.bfloat16)
                        g0, g1 = plsc.unpack(
                            gv, format=plsc.PackFormat.INTERLEAVED,
                            preferred_element_type=jnp.float32)
                        b0, b1 = plsc.unpack(
                            bv, format=plsc.PackFormat.INTERLEAVED,
                            preferred_element_type=jnp.float32)
                        x0, x1 = plsc.unpack(
                            xv, format=plsc.PackFormat.INTERLEAVED,
                            preferred_element_type=jnp.float32)
                        msgbuf[b, e, pl.ds(32 * j, 16)] = jnp.maximum(
                            g0 * x0 + b0, 0.0)
                        msgbuf[b, e, pl.ds(32 * j + 16, 16)] = jnp.maximum(
                            g1 * x1 + b1, 0.0)

                pltpu.async_copy(
                    msgbuf.at[b], accum.at[dstb.at[b]], ss[b], add=True)

                @pl.when(ci + 3 < NCHE)
                def _():
                    fire(ci + 3, b)

            return 0

        lax.fori_loop(0, NCHE // 3, body, 0)
        for b in range(NCHE - 3 * (NCHE // 3)):
            ci = 3 * (NCHE // 3) + b
            pltpu.make_async_copy(
                xl_hbm.at[pl.ds(0, CH)], xbuf.at[b], gx[b]).wait()
            pltpu.make_async_copy(
                gb_hbm.at[pl.ds(0, CH)], gbbuf.at[b], gg[b]).wait()
            for j in range(CH // 16):
                dstb[b, pl.ds(16 * j, 16)] = lax.shift_right_logical(
                    idst[pl.ds(ci * CH + 16 * j, 16)], 3)
            pltpu.make_async_copy(
                msgbuf.at[b], accum.at[pl.ds(0, CH)], ss[b]).wait()

            @plsc.parallel_loop(0, CH, unroll=2)
            def _(e):
                for j in range(DH // 32):
                    gv = plsc.bitcast(
                        gbbuf[b, e, pl.ds(DH // 2 + 16 * j, 16)], jnp.bfloat16)
                    bv = plsc.bitcast(
                        gbbuf[b, e, pl.ds(16 * j, 16)], jnp.bfloat16)
                    xv = plsc.bitcast(
                        xbuf[b, e, pl.ds(16 * j, 16)], jnp.bfloat16)
                    g0, g1 = plsc.unpack(
                        gv, format=plsc.PackFormat.INTERLEAVED,
                        preferred_element_type=jnp.float32)
                    b0, b1 = plsc.unpack(
                        bv, format=plsc.PackFormat.INTERLEAVED,
                        preferred_element_type=jnp.float32)
                    x0, x1 = plsc.unpack(
                        xv, format=plsc.PackFormat.INTERLEAVED,
                        preferred_element_type=jnp.float32)
                    msgbuf[b, e, pl.ds(32 * j, 16)] = jnp.maximum(
                        g0 * x0 + b0, 0.0)
                    msgbuf[b, e, pl.ds(32 * j + 16, 16)] = jnp.maximum(
                        g1 * x1 + b1, 0.0)

            pltpu.async_copy(
                msgbuf.at[b], accum.at[dstb.at[b]], ss[b], add=True)
        for b in range(3):
            pltpu.make_async_copy(
                msgbuf.at[b], accum.at[pl.ds(0, CH)], ss[b]).wait()
        plsc.subcore_barrier()
        pltpu.sync_copy(accum.at[pl.ds(rbase, RPT)],
                        out_hbm.at[pl.ds(rbase, RPT), pl.ds(c * DH, DH)])

    return k(xl_tab, gb_tab, srcr8, dstr8)


def _lohi(w64):
    lo = jnp.concatenate([w64[..., 0:16], w64[..., 32:48]], axis=-1)
    hi = jnp.concatenate([w64[..., 16:32], w64[..., 48:64]], axis=-1)
    return lo, hi


def kernel(x, edge_index, edge_type, mask, params):
    src = edge_index[0].astype(jnp.int32)
    dst = edge_index[1].astype(jnp.int32)
    rt = edge_type.astype(jnp.int32)
    srcr8 = src * (2 * R) + rt
    dstr8 = dst * (2 * R) + rt

    cnt = _count_sc(dstr8)
    cnt2 = cnt.reshape(NC, N, D)
    inv_n = _prep_inv(cnt2)

    p = params
    h = _enc(x, p["enc"]["W"].T, p["enc"]["b"])

    gp = p["gru_s"]
    wa = jnp.concatenate(
        [gp["z0"]["W"].T, gp["r0"]["W"].T, gp["h0"]["W"].T], axis=1)
    wx = jnp.concatenate([gp["z1"]["W"].T, gp["r1"]["W"].T], axis=1)
    ws = jnp.concatenate(
        [gp["z2"]["W"].T, gp["r2"]["W"].T, gp["h2"]["W"].T], axis=1)
    wh1 = gp["h1"]["W"].T
    gbias = jnp.concatenate([
        gp["z0"]["b"] + gp["z1"]["b"] + gp["z2"]["b"],
        gp["r0"]["b"] + gp["r1"]["b"] + gp["r2"]["b"],
        gp["h0"]["b"] + gp["h1"]["b"] + gp["h2"]["b"],
    ]).reshape(1, 3 * D)

    for step in range(NSTEP):
        cp = p["conv"][step]
        xl_lo, xl_hi = [], []
        for c in range(NC):
            for r in range(R):
                lo, hi = _lohi(cp["lins"][r]["W"].T[:, c * DH:(c + 1) * DH])
                xl_lo.append(lo)
                xl_hi.append(hi)
        gb_lo, gb_hi, bias_lo, bias_hi = [], [], [], []
        for c in range(NC):
            for r in range(R):
                wt = cp["films"][r]["W"].T
                bb = cp["films"][r]["b"]
                for piece in (wt[:, c * DH:(c + 1) * DH],
                              wt[:, D + c * DH:D + (c + 1) * DH]):
                    lo, hi = _lohi(piece)
                    gb_lo.append(lo)
                    gb_hi.append(hi)
                for piece in (bb[c * DH:(c + 1) * DH],
                              bb[D + c * DH:D + (c + 1) * DH]):
                    lo, hi = _lohi(piece)
                    bias_lo.append(lo)
                    bias_hi.append(hi)
        w_cat = jnp.concatenate(
            [cp["lin_skip"]["W"].T, cp["film_skip"]["W"].T]
            + xl_lo + xl_hi + gb_lo + gb_hi,
            axis=1)
        bias_cat = jnp.concatenate(
            bias_lo + bias_hi).reshape(1, 8 * D)

        skip, xl, gb = _pre(h, w_cat, bias_cat, inv_n)
        xl_tab = xl.reshape(2 * NR, DH // 2)
        gb_tab = gb.reshape(2 * NR, DH)

        part = _edges_sc(xl_tab, gb_tab, srcr8, dstr8)
        q = _attn(h.reshape(B, L, D), mask)
        sn = jnp.repeat(q, L, axis=0)
        h = _gru(h, skip, part, sn, wa, wx, ws, wh1, gbias)

    return h.reshape(B, L, D)
